# baseline (speedup 1.0000x reference)
import jax
import jax.numpy as jnp
from jax.experimental import pallas as pl
from jax.experimental.pallas import tpu as pltpu

N = 2708
D = 1433
NOISE = 5
HID = 128
OUT = 128
NP = 2816
DP = 1536

BM = 256
GAT_BM = 256
GAT_BN = 256
F32 = jnp.float32


def _pad2(a, r, c):
    return jnp.pad(a, ((0, r - a.shape[0]), (0, c - a.shape[1])))


def _dot(a, b):
    return jnp.dot(a, b, preferred_element_type=F32)


def _k1_body(x_ref, w_ref, nn0_ref, nn1_ref, wn_ref, b1_ref, b1v_ref, o_ref):
    acc = _dot(x_ref[...], w_ref[...])
    xa = acc[:, :HID] + b1_ref[...]
    g1 = acc[:, HID:] + b1v_ref[...]
    h0 = xa + _dot(nn0_ref[...], wn_ref[...])
    h1 = xa + _dot(nn1_ref[...], wn_ref[...])
    o_ref[...] = jnp.concatenate([h0, h1, g1], axis=1)


def _k1(x_p, wcat, nn0, nn1, wn, b1, b1v):
    grid = (NP // BM,)
    return pl.pallas_call(
        _k1_body,
        grid=grid,
        in_specs=[
            pl.BlockSpec((BM, DP), lambda i: (i, 0)),
            pl.BlockSpec((DP, 2 * HID), lambda i: (0, 0)),
            pl.BlockSpec((BM, 128), lambda i: (i, 0)),
            pl.BlockSpec((BM, 128), lambda i: (i, 0)),
            pl.BlockSpec((128, HID), lambda i: (0, 0)),
            pl.BlockSpec((1, HID), lambda i: (0, 0)),
            pl.BlockSpec((1, HID), lambda i: (0, 0)),
        ],
        out_specs=pl.BlockSpec((BM, 3 * HID), lambda i: (i, 0)),
        out_shape=jax.ShapeDtypeStruct((NP, 3 * HID), F32),
    )(x_p, wcat, nn0, nn1, wn, b1, b1v)


def _k2_body(adj_ref, s1_ref, wbd_ref, b2_ref, o_ref):
    t = _dot(adj_ref[...], s1_ref[...])
    o_ref[...] = _dot(jnp.maximum(t, 0.0), wbd_ref[...]) + b2_ref[...]


def _k2(adj_p, s1, wbd, b2cat):
    grid = (NP // BM,)
    return pl.pallas_call(
        _k2_body,
        grid=grid,
        in_specs=[
            pl.BlockSpec((BM, NP), lambda i: (i, 0)),
            pl.BlockSpec((NP, 3 * HID), lambda i: (0, 0)),
            pl.BlockSpec((3 * HID, 3 * HID), lambda i: (0, 0)),
            pl.BlockSpec((1, 3 * HID), lambda i: (0, 0)),
        ],
        out_specs=pl.BlockSpec((BM, 3 * HID), lambda i: (i, 0)),
        out_shape=jax.ShapeDtypeStruct((NP, 3 * HID), F32),
    )(adj_p, s1, wbd, b2cat)


def _k3_body(adj_ref, s2_ref, eps_ref, m_ref, z_ref, sig_ref):
    m = _dot(adj_ref[...], s2_ref[...])
    m_ref[...] = m
    sig = jnp.exp(0.5 * m[:, 2 * HID:])
    sig_ref[...] = sig
    z_ref[...] = m[:, :HID] + eps_ref[...] * sig


def _k3(adj_p, s2, eps0):
    grid = (NP // BM,)
    return pl.pallas_call(
        _k3_body,
        grid=grid,
        in_specs=[
            pl.BlockSpec((BM, NP), lambda i: (i, 0)),
            pl.BlockSpec((NP, 3 * HID), lambda i: (0, 0)),
            pl.BlockSpec((BM, HID), lambda i: (i, 0)),
        ],
        out_specs=[
            pl.BlockSpec((BM, 3 * HID), lambda i: (i, 0)),
            pl.BlockSpec((BM, HID), lambda i: (i, 0)),
            pl.BlockSpec((BM, HID), lambda i: (i, 0)),
        ],
        out_shape=[
            jax.ShapeDtypeStruct((NP, 3 * HID), F32),
            jax.ShapeDtypeStruct((NP, HID), F32),
            jax.ShapeDtypeStruct((NP, HID), F32),
        ],
    )(adj_p, s2, eps0)


def _k4_body(xt_ref, w_ref, an0_ref, an1_ref, wan_ref, b1_ref, b1v_ref,
             w2mu_ref, b2mu_ref, w2v_ref, b2v_ref, aeps_ref,
             m0_ref, m1_ref, lv_ref, sig_ref, za_ref):
    a = _dot(xt_ref[...], w_ref[...])
    base = a[:, :HID] + b1_ref[...]
    n0 = _dot(an0_ref[...], wan_ref[...])
    n1 = _dot(an1_ref[...], wan_ref[...])
    u0 = jnp.maximum(base + n0, 0.0)
    u1 = jnp.maximum(base + n1, 0.0)
    v = jnp.maximum(a[:, HID:] + b1v_ref[...], 0.0)
    m0 = _dot(u0, w2mu_ref[...]) + b2mu_ref[...]
    m1 = _dot(u1, w2mu_ref[...]) + b2mu_ref[...]
    lv = _dot(v, w2v_ref[...]) + b2v_ref[...]
    sig = jnp.exp(0.5 * lv)
    m0_ref[...] = m0
    m1_ref[...] = m1
    lv_ref[...] = lv
    sig_ref[...] = sig
    za_ref[...] = m0 + aeps_ref[...] * sig


def _k4(xt_p, wacat, an0, an1, wan, ab1, ab1v, w2mu, b2mu, w2v, b2v, aeps0):
    grid = (DP // BM,)
    spec_row = pl.BlockSpec((BM, 128), lambda i: (i, 0))
    spec_w = pl.BlockSpec((128, 128), lambda i: (0, 0))
    spec_b = pl.BlockSpec((1, 128), lambda i: (0, 0))
    return pl.pallas_call(
        _k4_body,
        grid=grid,
        in_specs=[
            pl.BlockSpec((BM, NP), lambda i: (i, 0)),
            pl.BlockSpec((NP, 2 * HID), lambda i: (0, 0)),
            spec_row, spec_row, spec_w, spec_b, spec_b,
            spec_w, spec_b, spec_w, spec_b, spec_row,
        ],
        out_specs=[spec_row] * 5,
        out_shape=[jax.ShapeDtypeStruct((DP, HID), F32)] * 5,
    )(xt_p, wacat, an0, an1, wan, ab1, ab1v, w2mu, b2mu, w2v, b2v, aeps0)


def _k5_body(zl_ref, zr_ref, o_ref):
    o_ref[...] = jax.lax.dot_general(
        zl_ref[...], zr_ref[...], (((1,), (1,)), ((), ())),
        preferred_element_type=F32)


def _k5(z_u):
    grid = (NP // BM, NP // BM)
    return pl.pallas_call(
        _k5_body,
        grid=grid,
        in_specs=[
            pl.BlockSpec((BM, HID), lambda i, j: (i, 0)),
            pl.BlockSpec((BM, HID), lambda i, j: (j, 0)),
        ],
        out_specs=pl.BlockSpec((BM, BM), lambda i, j: (i, j)),
        out_shape=jax.ShapeDtypeStruct((NP, NP), F32),
    )(z_u, z_u)


def _k6_body(x_ref, za_ref, o_ref):
    xz = _dot(x_ref[...], za_ref[...])
    rs = jnp.sum(jnp.abs(x_ref[...]), axis=1, keepdims=True)
    o_ref[...] = xz / jnp.maximum(rs, 1e-12)


def _k6(x_p, za_p):
    grid = (NP // BM,)
    return pl.pallas_call(
        _k6_body,
        grid=grid,
        in_specs=[
            pl.BlockSpec((BM, DP), lambda i: (i, 0)),
            pl.BlockSpec((DP, HID), lambda i: (0, 0)),
        ],
        out_specs=pl.BlockSpec((BM, HID), lambda i: (i, 0)),
        out_shape=jax.ShapeDtypeStruct((NP, HID), F32),
    )(x_p, za_p)


HBN = 512


def _k7_body(cf_ref, w_ref, a2_ref, h_ref, ee_ref):
    h = _dot(cf_ref[...], w_ref[...])
    h_ref[...] = h

    @pl.when(pl.program_id(1) == 0)
    def _():
        ee_ref[...] = jnp.zeros_like(ee_ref)

    ee_ref[...] += _dot(h, a2_ref[...])


def _k7(cf, decw_p, a2):
    grid = (NP // BM, DP // HBN)
    return pl.pallas_call(
        _k7_body,
        grid=grid,
        in_specs=[
            pl.BlockSpec((BM, 2 * HID), lambda i, j: (i, 0)),
            pl.BlockSpec((2 * HID, HBN), lambda i, j: (0, j)),
            pl.BlockSpec((HBN, 128), lambda i, j: (j, 0)),
        ],
        out_specs=[
            pl.BlockSpec((BM, HBN), lambda i, j: (i, j)),
            pl.BlockSpec((BM, 128), lambda i, j: (i, 0)),
        ],
        out_shape=[
            jax.ShapeDtypeStruct((NP, DP), F32),
            jax.ShapeDtypeStruct((NP, 128), F32),
        ],
    )(cf, decw_p, a2)


def _k8_body(ee_ref, elt_ref, adj_ref, h_ref, b_ref, o_ref,
             acc_ref, m_ref, l_ref):
    j = pl.program_id(1)
    nj = pl.num_programs(1)

    @pl.when(j == 0)
    def _():
        acc_ref[...] = jnp.zeros_like(acc_ref)
        m_ref[...] = jnp.full_like(m_ref, -1e30)
        l_ref[...] = jnp.zeros_like(l_ref)

    er = ee_ref[...][:, 1:2]
    el = elt_ref[...]
    e = er + el
    e = jnp.where(e > 0, e, 0.2 * e)
    e = jnp.where(adj_ref[...] > 0, e, -1e9)

    m_prev = m_ref[...]
    m_new = jnp.maximum(m_prev, jnp.max(e, axis=1, keepdims=True))
    p = jnp.exp(e - m_new)
    corr = jnp.exp(m_prev - m_new)
    l_ref[...] = l_ref[...] * corr + jnp.sum(p, axis=1, keepdims=True)
    acc_ref[...] = acc_ref[...] * corr + _dot(p, h_ref[...])
    m_ref[...] = m_new

    @pl.when(j == nj - 1)
    def _():
        o_ref[...] = acc_ref[...] / l_ref[...] + b_ref[...]


def _k8(ee, elt, adj_p, h, decb):
    grid = (NP // GAT_BM, NP // GAT_BN)
    return pl.pallas_call(
        _k8_body,
        grid=grid,
        in_specs=[
            pl.BlockSpec((GAT_BM, 128), lambda i, j: (i, 0)),
            pl.BlockSpec((1, GAT_BN), lambda i, j: (0, j)),
            pl.BlockSpec((GAT_BM, GAT_BN), lambda i, j: (i, j)),
            pl.BlockSpec((GAT_BN, DP), lambda i, j: (j, 0)),
            pl.BlockSpec((1, DP), lambda i, j: (0, 0)),
        ],
        out_specs=pl.BlockSpec((GAT_BM, DP), lambda i, j: (i, 0)),
        out_shape=jax.ShapeDtypeStruct((NP, DP), F32),
        scratch_shapes=[
            pltpu.VMEM((GAT_BM, DP), F32),
            pltpu.VMEM((GAT_BM, 1), F32),
            pltpu.VMEM((GAT_BM, 1), F32),
        ],
    )(ee, elt, adj_p, h, decb)


def kernel(graph, x, nmu_W1, nmu_b1, nmu_W2, nmu_b2, nvar_W1, nvar_b1,
           nvar_W2, nvar_b2, amu_W1, amu_b1, amu_W2, amu_b2, avar_W1,
           avar_b1, avar_W2, avar_b2, dec_W, dec_al, dec_ar, dec_b):
    f32 = F32
    rk = jax.random.key(7)
    r = jax.random.split(rk, 4)
    node_noise = jax.random.bernoulli(r[0], 0.5, (N, 2, NOISE)).astype(f32)
    attr_noise = jax.random.bernoulli(r[1], 0.5, (D, 2, NOISE)).astype(f32)
    node_eps = jax.random.normal(r[2], (N, 1, OUT), dtype=f32)
    attr_eps = jax.random.normal(r[3], (D, 1, 128), dtype=f32)

    adj_p = _pad2(graph, NP, NP)
    x_p = _pad2(x, NP, DP)
    xt_p = _pad2(x.T, DP, NP)

    wcat = _pad2(jnp.concatenate([nmu_W1[NOISE:], nvar_W1], axis=1), DP, 2 * HID)
    wn = _pad2(nmu_W1[:NOISE], 128, HID)
    nn0 = _pad2(node_noise[:, 0, :], NP, 128)
    nn1 = _pad2(node_noise[:, 1, :], NP, 128)
    b1 = nmu_b1.reshape(1, HID)
    b1v = nvar_b1.reshape(1, HID)

    wbd = jnp.zeros((3 * HID, 3 * HID), f32)
    wbd = wbd.at[:HID, :HID].set(nmu_W2)
    wbd = wbd.at[HID:2 * HID, HID:2 * HID].set(nmu_W2)
    wbd = wbd.at[2 * HID:, 2 * HID:].set(nvar_W2)
    b2cat = jnp.concatenate([nmu_b2, nmu_b2, nvar_b2]).reshape(1, 3 * HID)

    eps0 = _pad2(node_eps[:, 0, :], NP, HID)

    wacat = _pad2(jnp.concatenate([amu_W1[NOISE:], avar_W1], axis=1), NP, 2 * HID)
    wan = _pad2(amu_W1[:NOISE], 128, 128)
    an0 = _pad2(attr_noise[:, 0, :], DP, 128)
    an1 = _pad2(attr_noise[:, 1, :], DP, 128)
    aeps0 = _pad2(attr_eps[:, 0, :], DP, 128)

    s1 = _k1(x_p, wcat, nn0, nn1, wn, b1, b1v)
    s2 = _k2(adj_p, s1, wbd, b2cat)
    m_all, z_u, sigma_n = _k3(adj_p, s2, eps0)

    am0, am1, alv, asig, z_a = _k4(
        xt_p, wacat, an0, an1, wan,
        amu_b1.reshape(1, 128), avar_b1.reshape(1, 128),
        amu_W2, amu_b2.reshape(1, 128), avar_W2, avar_b2.reshape(1, 128),
        aeps0)

    links = _k5(z_u)
    fine = _k6(x_p, z_a)
    cf = jnp.concatenate([z_u, fine], axis=1)
    decw_p = _pad2(dec_W, 2 * HID, DP)
    a2 = jnp.zeros((DP, 128), f32)
    a2 = a2.at[:D, 0].set(dec_al)
    a2 = a2.at[:D, 1].set(dec_ar)
    h, ee = _k7(cf, decw_p, a2)
    elt = ee[:, 0:1].T
    decb = jnp.pad(dec_b, (0, DP - D)).reshape(1, DP)
    out_a = _k8(ee, elt, adj_p, h, decb)

    node_mu0 = m_all[:N, :HID]
    node_mu1 = m_all[:N, HID:2 * HID]
    node_logv = m_all[:N, 2 * HID:]
    sig_n = sigma_n[:N]
    zu = z_u[:N]

    merged_node_mu = jnp.stack([node_mu1, node_mu0], axis=1)[:, None, :, :]
    merged_node_sigma = jnp.repeat(sig_n[:, None, None, :], 2, axis=2)
    merged_node_z = jnp.repeat(zu[:, None, None, :], 2, axis=2)
    node_logv_iw = node_logv[:, None, :]
    node_z_iw = zu[:, None, :]

    attr_mu0 = am0[:D]
    attr_mu1 = am1[:D]
    attr_logv = alv[:D]
    sig_a = asig[:D]
    za = z_a[:D]

    merged_attr_mu = jnp.stack([attr_mu1, attr_mu0], axis=1)[:, None, :, :]
    merged_attr_sigma = jnp.repeat(sig_a[:, None, None, :], 2, axis=2)
    merged_attr_z = jnp.repeat(za[:, None, None, :], 2, axis=2)
    attr_logv_iw = attr_logv[:, None, :]
    attr_z_iw = za[:, None, :]

    reconstruct_node_logits = links[:N, :N, None]
    reconstruct_attr_logits = out_a[:N, :D, None]

    return (merged_node_mu, merged_node_sigma, merged_node_z, node_logv_iw,
            node_z_iw, merged_attr_mu, merged_attr_sigma, merged_attr_z,
            attr_logv_iw, attr_z_iw, reconstruct_node_logits,
            reconstruct_attr_logits, node_mu0, attr_mu0)


# speedup vs baseline: 1.3763x; 1.3763x over previous
import jax
import jax.numpy as jnp
from jax.experimental import pallas as pl
from jax.experimental.pallas import tpu as pltpu

N = 2708
D = 1433
NOISE = 5
HID = 128
OUT = 128
F32 = jnp.float32

BM = 256
NBLK = 11
DBLK = 6
HBN = 512
HJ = 3
GAT_BM = 256
GAT_BN = 256

_rk = jax.random.key(7)
_r = jax.random.split(_rk, 4)
NODE_NOISE = jax.random.bernoulli(_r[0], 0.5, (N, 2, NOISE)).astype(F32)
ATTR_NOISE = jax.random.bernoulli(_r[1], 0.5, (D, 2, NOISE)).astype(F32)
NODE_EPS0 = jax.random.normal(_r[2], (N, 1, OUT), dtype=F32)[:, 0, :]
ATTR_EPS0 = jax.random.normal(_r[3], (D, 1, 128), dtype=F32)[:, 0, :]


def _dot(a, b):
    return jnp.dot(a, b, preferred_element_type=F32)


def _dot0(a, b):
    return jax.lax.dot_general(a, b, (((0,), (0,)), ((), ())),
                               preferred_element_type=F32)


def _k1_body(x_ref, w_ref, nn0_ref, nn1_ref, wn_ref, b1_ref, b1v_ref, o_ref):
    acc = _dot(x_ref[...], w_ref[...])
    xa = acc[:, :HID] + b1_ref[...]
    g1 = acc[:, HID:] + b1v_ref[...]
    h0 = xa + _dot(nn0_ref[...], wn_ref[...])
    h1 = xa + _dot(nn1_ref[...], wn_ref[...])
    o_ref[...] = jnp.concatenate([h0, h1, g1], axis=1)


def _k1(x, wcat, nn0, nn1, wn, b1, b1v):
    return pl.pallas_call(
        _k1_body,
        grid=(NBLK,),
        in_specs=[
            pl.BlockSpec((BM, D), lambda i: (i, 0)),
            pl.BlockSpec((D, 2 * HID), lambda i: (0, 0)),
            pl.BlockSpec((BM, NOISE), lambda i: (i, 0)),
            pl.BlockSpec((BM, NOISE), lambda i: (i, 0)),
            pl.BlockSpec((NOISE, HID), lambda i: (0, 0)),
            pl.BlockSpec((1, HID), lambda i: (0, 0)),
            pl.BlockSpec((1, HID), lambda i: (0, 0)),
        ],
        out_specs=pl.BlockSpec((BM, 3 * HID), lambda i: (i, 0)),
        out_shape=jax.ShapeDtypeStruct((N, 3 * HID), F32),
    )(x, wcat, nn0, nn1, wn, b1, b1v)


def _k2_body(adj_ref, s1_ref, wbd_ref, b2_ref, o_ref):
    t = _dot(adj_ref[...], s1_ref[...])
    o_ref[...] = _dot(jnp.maximum(t, 0.0), wbd_ref[...]) + b2_ref[...]


def _k2(adj, s1, wbd, b2cat):
    return pl.pallas_call(
        _k2_body,
        grid=(NBLK,),
        in_specs=[
            pl.BlockSpec((BM, N), lambda i: (i, 0)),
            pl.BlockSpec((N, 3 * HID), lambda i: (0, 0)),
            pl.BlockSpec((3 * HID, 3 * HID), lambda i: (0, 0)),
            pl.BlockSpec((1, 3 * HID), lambda i: (0, 0)),
        ],
        out_specs=pl.BlockSpec((BM, 3 * HID), lambda i: (i, 0)),
        out_shape=jax.ShapeDtypeStruct((N, 3 * HID), F32),
    )(adj, s1, wbd, b2cat)


def _k3_body(adj_ref, s2_ref, eps_ref, m_ref, z_ref, sig_ref):
    m = _dot(adj_ref[...], s2_ref[...])
    m_ref[...] = m
    sig = jnp.exp(0.5 * m[:, 2 * HID:])
    sig_ref[...] = sig
    z_ref[...] = m[:, :HID] + eps_ref[...] * sig


def _k3(adj, s2, eps0):
    return pl.pallas_call(
        _k3_body,
        grid=(NBLK,),
        in_specs=[
            pl.BlockSpec((BM, N), lambda i: (i, 0)),
            pl.BlockSpec((N, 3 * HID), lambda i: (0, 0)),
            pl.BlockSpec((BM, HID), lambda i: (i, 0)),
        ],
        out_specs=[
            pl.BlockSpec((BM, 3 * HID), lambda i: (i, 0)),
            pl.BlockSpec((BM, HID), lambda i: (i, 0)),
            pl.BlockSpec((BM, HID), lambda i: (i, 0)),
        ],
        out_shape=[
            jax.ShapeDtypeStruct((N, 3 * HID), F32),
            jax.ShapeDtypeStruct((N, HID), F32),
            jax.ShapeDtypeStruct((N, HID), F32),
        ],
    )(adj, s2, eps0)


def _k4_body(x_ref, w_ref, an0_ref, an1_ref, wan_ref, b1_ref, b1v_ref,
             w2mu_ref, b2mu_ref, w2v_ref, b2v_ref, aeps_ref,
             m0_ref, m1_ref, lv_ref, sig_ref, za_ref):
    a = _dot0(x_ref[...], w_ref[...])
    base = a[:, :HID] + b1_ref[...]
    n0 = _dot(an0_ref[...], wan_ref[...])
    n1 = _dot(an1_ref[...], wan_ref[...])
    u0 = jnp.maximum(base + n0, 0.0)
    u1 = jnp.maximum(base + n1, 0.0)
    v = jnp.maximum(a[:, HID:] + b1v_ref[...], 0.0)
    m0 = _dot(u0, w2mu_ref[...]) + b2mu_ref[...]
    m1 = _dot(u1, w2mu_ref[...]) + b2mu_ref[...]
    lv = _dot(v, w2v_ref[...]) + b2v_ref[...]
    sig = jnp.exp(0.5 * lv)
    m0_ref[...] = m0
    m1_ref[...] = m1
    lv_ref[...] = lv
    sig_ref[...] = sig
    za_ref[...] = m0 + aeps_ref[...] * sig


def _k4(x, wacat, an0, an1, wan, ab1, ab1v, w2mu, b2mu, w2v, b2v, aeps0):
    spec_row = pl.BlockSpec((BM, 128), lambda i: (i, 0))
    spec_n = pl.BlockSpec((BM, NOISE), lambda i: (i, 0))
    spec_w = pl.BlockSpec((128, 128), lambda i: (0, 0))
    spec_wn = pl.BlockSpec((NOISE, 128), lambda i: (0, 0))
    spec_b = pl.BlockSpec((1, 128), lambda i: (0, 0))
    return pl.pallas_call(
        _k4_body,
        grid=(DBLK,),
        in_specs=[
            pl.BlockSpec((N, BM), lambda i: (0, i)),
            pl.BlockSpec((N, 2 * HID), lambda i: (0, 0)),
            spec_n, spec_n, spec_wn, spec_b, spec_b,
            spec_w, spec_b, spec_w, spec_b, spec_row,
        ],
        out_specs=[spec_row] * 5,
        out_shape=[jax.ShapeDtypeStruct((D, HID), F32)] * 5,
    )(x, wacat, an0, an1, wan, ab1, ab1v, w2mu, b2mu, w2v, b2v, aeps0)


def _k5_body(zl_ref, zr_ref, o_ref):
    o_ref[...] = jax.lax.dot_general(
        zl_ref[...], zr_ref[...], (((1,), (1,)), ((), ())),
        preferred_element_type=F32)


def _k5(z_u):
    return pl.pallas_call(
        _k5_body,
        grid=(NBLK, NBLK),
        in_specs=[
            pl.BlockSpec((BM, HID), lambda i, j: (i, 0)),
            pl.BlockSpec((BM, HID), lambda i, j: (j, 0)),
        ],
        out_specs=pl.BlockSpec((BM, BM), lambda i, j: (i, j)),
        out_shape=jax.ShapeDtypeStruct((N, N), F32),
    )(z_u, z_u)


def _k6_body(x_ref, za_ref, o_ref):
    xz = _dot(x_ref[...], za_ref[...])
    rs = jnp.sum(jnp.abs(x_ref[...]), axis=1, keepdims=True)
    o_ref[...] = xz / jnp.maximum(rs, 1e-12)


def _k6(x, za):
    return pl.pallas_call(
        _k6_body,
        grid=(NBLK,),
        in_specs=[
            pl.BlockSpec((BM, D), lambda i: (i, 0)),
            pl.BlockSpec((D, HID), lambda i: (0, 0)),
        ],
        out_specs=pl.BlockSpec((BM, HID), lambda i: (i, 0)),
        out_shape=jax.ShapeDtypeStruct((N, HID), F32),
    )(x, za)


def _k7_body(cf_ref, w_ref, a2_ref, h_ref, ee_ref):
    i = pl.program_id(0)
    j = pl.program_id(1)
    h = _dot(cf_ref[...], w_ref[...])
    row = i * BM + jax.lax.broadcasted_iota(jnp.int32, h.shape, 0)
    col = j * HBN + jax.lax.broadcasted_iota(jnp.int32, h.shape, 1)
    h = jnp.where((row < N) & (col < D), h, 0.0)
    h_ref[...] = h

    @pl.when(j == 0)
    def _():
        ee_ref[...] = jnp.zeros_like(ee_ref)

    ee_ref[...] += _dot(h, a2_ref[...])


def _k7(cf, dec_w, a2):
    return pl.pallas_call(
        _k7_body,
        grid=(NBLK, HJ),
        in_specs=[
            pl.BlockSpec((BM, 2 * HID), lambda i, j: (i, 0)),
            pl.BlockSpec((2 * HID, HBN), lambda i, j: (0, j)),
            pl.BlockSpec((HBN, 128), lambda i, j: (j, 0)),
        ],
        out_specs=[
            pl.BlockSpec((BM, HBN), lambda i, j: (i, j)),
            pl.BlockSpec((BM, 128), lambda i, j: (i, 0)),
        ],
        out_shape=[
            jax.ShapeDtypeStruct((NBLK * BM, D), F32),
            jax.ShapeDtypeStruct((NBLK * BM, 128), F32),
        ],
    )(cf, dec_w, a2)


def _k8_body(ee_ref, elt_ref, adj_ref, h_ref, b_ref, o_ref,
             acc_ref, m_ref, l_ref):
    j = pl.program_id(1)
    nj = pl.num_programs(1)

    @pl.when(j == 0)
    def _():
        acc_ref[...] = jnp.zeros_like(acc_ref)
        m_ref[...] = jnp.full_like(m_ref, -1e30)
        l_ref[...] = jnp.zeros_like(l_ref)

    er = ee_ref[...][:, 1:2]
    el = elt_ref[...]
    e = er + el
    e = jnp.where(e > 0, e, 0.2 * e)
    e = jnp.where(adj_ref[...] > 0, e, -1e9)
    col = j * GAT_BN + jax.lax.broadcasted_iota(jnp.int32, e.shape, 1)
    e = jnp.where(col < N, e, -1e9)

    m_prev = m_ref[...]
    m_new = jnp.maximum(m_prev, jnp.max(e, axis=1, keepdims=True))
    p = jnp.exp(e - m_new)
    corr = jnp.exp(m_prev - m_new)
    l_ref[...] = l_ref[...] * corr + jnp.sum(p, axis=1, keepdims=True)
    acc_ref[...] = acc_ref[...] * corr + _dot(p, h_ref[...])
    m_ref[...] = m_new

    @pl.when(j == nj - 1)
    def _():
        o_ref[...] = acc_ref[...] / l_ref[...] + b_ref[...]


def _k8(ee, elt, adj, h, decb):
    return pl.pallas_call(
        _k8_body,
        grid=(NBLK, NBLK),
        in_specs=[
            pl.BlockSpec((GAT_BM, 128), lambda i, j: (i, 0)),
            pl.BlockSpec((1, GAT_BN), lambda i, j: (0, j)),
            pl.BlockSpec((GAT_BM, GAT_BN), lambda i, j: (i, j)),
            pl.BlockSpec((GAT_BN, D), lambda i, j: (j, 0)),
            pl.BlockSpec((1, D), lambda i, j: (0, 0)),
        ],
        out_specs=pl.BlockSpec((GAT_BM, D), lambda i, j: (i, 0)),
        out_shape=jax.ShapeDtypeStruct((N, D), F32),
        scratch_shapes=[
            pltpu.VMEM((GAT_BM, D), F32),
            pltpu.VMEM((GAT_BM, 1), F32),
            pltpu.VMEM((GAT_BM, 1), F32),
        ],
    )(ee, elt, adj, h, decb)


def kernel(graph, x, nmu_W1, nmu_b1, nmu_W2, nmu_b2, nvar_W1, nvar_b1,
           nvar_W2, nvar_b2, amu_W1, amu_b1, amu_W2, amu_b2, avar_W1,
           avar_b1, avar_W2, avar_b2, dec_W, dec_al, dec_ar, dec_b):
    f32 = F32
    nn0 = NODE_NOISE[:, 0, :]
    nn1 = NODE_NOISE[:, 1, :]
    an0 = ATTR_NOISE[:, 0, :]
    an1 = ATTR_NOISE[:, 1, :]

    wcat = jnp.concatenate([nmu_W1[NOISE:], nvar_W1], axis=1)
    wn = nmu_W1[:NOISE]
    b1 = nmu_b1.reshape(1, HID)
    b1v = nvar_b1.reshape(1, HID)

    wbd = jnp.zeros((3 * HID, 3 * HID), f32)
    wbd = wbd.at[:HID, :HID].set(nmu_W2)
    wbd = wbd.at[HID:2 * HID, HID:2 * HID].set(nmu_W2)
    wbd = wbd.at[2 * HID:, 2 * HID:].set(nvar_W2)
    b2cat = jnp.concatenate([nmu_b2, nmu_b2, nvar_b2]).reshape(1, 3 * HID)

    wacat = jnp.concatenate([amu_W1[NOISE:], avar_W1], axis=1)
    wan = amu_W1[:NOISE]

    s1 = _k1(x, wcat, nn0, nn1, wn, b1, b1v)
    s2 = _k2(graph, s1, wbd, b2cat)
    m_all, z_u, sig_n = _k3(graph, s2, NODE_EPS0)

    am0, am1, alv, asig, z_a = _k4(
        x, wacat, an0, an1, wan,
        amu_b1.reshape(1, 128), avar_b1.reshape(1, 128),
        amu_W2, amu_b2.reshape(1, 128), avar_W2, avar_b2.reshape(1, 128),
        ATTR_EPS0)

    links = _k5(z_u)
    fine = _k6(x, z_a)
    cf = jnp.concatenate([z_u, fine], axis=1)
    a2 = jnp.zeros((HJ * HBN, 128), f32)
    a2 = a2.at[:D, 0].set(dec_al)
    a2 = a2.at[:D, 1].set(dec_ar)
    h, ee = _k7(cf, dec_W, a2)
    elt = ee[:, 0:1].T
    out_a = _k8(ee, elt, graph, h, dec_b.reshape(1, D))

    node_mu0 = m_all[:, :HID]
    node_mu1 = m_all[:, HID:2 * HID]
    node_logv = m_all[:, 2 * HID:]

    merged_node_mu = jnp.stack([node_mu1, node_mu0], axis=1)[:, None, :, :]
    merged_node_sigma = jnp.repeat(sig_n[:, None, None, :], 2, axis=2)
    merged_node_z = jnp.repeat(z_u[:, None, None, :], 2, axis=2)
    node_logv_iw = node_logv[:, None, :]
    node_z_iw = z_u[:, None, :]

    merged_attr_mu = jnp.stack([am1, am0], axis=1)[:, None, :, :]
    merged_attr_sigma = jnp.repeat(asig[:, None, None, :], 2, axis=2)
    merged_attr_z = jnp.repeat(z_a[:, None, None, :], 2, axis=2)
    attr_logv_iw = alv[:, None, :]
    attr_z_iw = z_a[:, None, :]

    reconstruct_node_logits = links[:, :, None]
    reconstruct_attr_logits = out_a[:, :, None]

    return (merged_node_mu, merged_node_sigma, merged_node_z, node_logv_iw,
            node_z_iw, merged_attr_mu, merged_attr_sigma, merged_attr_z,
            attr_logv_iw, attr_z_iw, reconstruct_node_logits,
            reconstruct_attr_logits, node_mu0, am0)


# speedup vs baseline: 1.6936x; 1.2305x over previous
import jax
import jax.numpy as jnp
from jax.experimental import pallas as pl
from jax.experimental.pallas import tpu as pltpu

N = 2708
D = 1433
NOISE = 5
HID = 128
OUT = 128
F32 = jnp.float32

BM = 256
NBLK = 11
DBLK = 6
HBN = 512
HJ = 3
GAT_BM = 256
GAT_BN = 1408
GAT_NJ = 2


def _rng_consts():
    rk = jax.random.key(7)
    r = jax.random.split(rk, 4)
    node_noise = jax.random.bernoulli(r[0], 0.5, (N, 2, NOISE)).astype(F32)
    attr_noise = jax.random.bernoulli(r[1], 0.5, (D, 2, NOISE)).astype(F32)
    node_eps0 = jax.random.normal(r[2], (N, 1, OUT), dtype=F32)[:, 0, :]
    attr_eps0 = jax.random.normal(r[3], (D, 1, 128), dtype=F32)[:, 0, :]
    return node_noise, attr_noise, node_eps0, attr_eps0


def _dot(a, b):
    return jnp.dot(a, b, preferred_element_type=F32)


def _dot0(a, b):
    return jax.lax.dot_general(a, b, (((0,), (0,)), ((), ())),
                               preferred_element_type=F32)


def _dot1(a, b):
    return jax.lax.dot_general(a, b, (((1,), (1,)), ((), ())),
                               preferred_element_type=F32)


def _k1_body(xt_ref, w_ref, nn0_ref, nn1_ref, wn_ref, b1_ref, b1v_ref, o_ref):
    acc = _dot0(xt_ref[...], w_ref[...])
    xa = acc[:, :HID] + b1_ref[...]
    g1 = acc[:, HID:] + b1v_ref[...]
    h0 = xa + _dot(nn0_ref[...], wn_ref[...])
    h1 = xa + _dot(nn1_ref[...], wn_ref[...])
    o_ref[...] = jnp.concatenate([h0, h1, g1], axis=1)


def _k1(xt, wcat, nn0, nn1, wn, b1, b1v):
    return pl.pallas_call(
        _k1_body,
        grid=(NBLK,),
        in_specs=[
            pl.BlockSpec((D, BM), lambda i: (0, i)),
            pl.BlockSpec((D, 2 * HID), lambda i: (0, 0)),
            pl.BlockSpec((BM, NOISE), lambda i: (i, 0)),
            pl.BlockSpec((BM, NOISE), lambda i: (i, 0)),
            pl.BlockSpec((NOISE, HID), lambda i: (0, 0)),
            pl.BlockSpec((1, HID), lambda i: (0, 0)),
            pl.BlockSpec((1, HID), lambda i: (0, 0)),
        ],
        out_specs=pl.BlockSpec((BM, 3 * HID), lambda i: (i, 0)),
        out_shape=jax.ShapeDtypeStruct((N, 3 * HID), F32),
    )(xt, wcat, nn0, nn1, wn, b1, b1v)


def _k2_body(adj_ref, s1_ref, wbd_ref, b2_ref, o_ref):
    t = _dot(adj_ref[...], s1_ref[...])
    o_ref[...] = _dot(jnp.maximum(t, 0.0), wbd_ref[...]) + b2_ref[...]


def _k2(adj, s1, wbd, b2cat):
    return pl.pallas_call(
        _k2_body,
        grid=(NBLK,),
        in_specs=[
            pl.BlockSpec((BM, N), lambda i: (i, 0)),
            pl.BlockSpec((N, 3 * HID), lambda i: (0, 0)),
            pl.BlockSpec((3 * HID, 3 * HID), lambda i: (0, 0)),
            pl.BlockSpec((1, 3 * HID), lambda i: (0, 0)),
        ],
        out_specs=pl.BlockSpec((BM, 3 * HID), lambda i: (i, 0)),
        out_shape=jax.ShapeDtypeStruct((N, 3 * HID), F32),
    )(adj, s1, wbd, b2cat)


def _k3_body(adj_ref, s2_ref, eps_ref, m_ref, z_ref, sig_ref):
    m = _dot(adj_ref[...], s2_ref[...])
    m_ref[...] = m
    sig = jnp.exp(0.5 * m[:, 2 * HID:])
    sig_ref[...] = sig
    z_ref[...] = m[:, :HID] + eps_ref[...] * sig


def _k3(adj, s2, eps0):
    return pl.pallas_call(
        _k3_body,
        grid=(NBLK,),
        in_specs=[
            pl.BlockSpec((BM, N), lambda i: (i, 0)),
            pl.BlockSpec((N, 3 * HID), lambda i: (0, 0)),
            pl.BlockSpec((BM, HID), lambda i: (i, 0)),
        ],
        out_specs=[
            pl.BlockSpec((BM, 3 * HID), lambda i: (i, 0)),
            pl.BlockSpec((BM, HID), lambda i: (i, 0)),
            pl.BlockSpec((BM, HID), lambda i: (i, 0)),
        ],
        out_shape=[
            jax.ShapeDtypeStruct((N, 3 * HID), F32),
            jax.ShapeDtypeStruct((N, HID), F32),
            jax.ShapeDtypeStruct((N, HID), F32),
        ],
    )(adj, s2, eps0)


def _k4_body(xt_ref, w_ref, an0_ref, an1_ref, wan_ref, b1_ref, b1v_ref,
             w2mu_ref, b2mu_ref, w2v_ref, b2v_ref, aeps_ref,
             m0_ref, m1_ref, lv_ref, sig_ref, za_ref):
    a = _dot(xt_ref[...], w_ref[...])
    base = a[:, :HID] + b1_ref[...]
    n0 = _dot(an0_ref[...], wan_ref[...])
    n1 = _dot(an1_ref[...], wan_ref[...])
    u0 = jnp.maximum(base + n0, 0.0)
    u1 = jnp.maximum(base + n1, 0.0)
    v = jnp.maximum(a[:, HID:] + b1v_ref[...], 0.0)
    m0 = _dot(u0, w2mu_ref[...]) + b2mu_ref[...]
    m1 = _dot(u1, w2mu_ref[...]) + b2mu_ref[...]
    lv = _dot(v, w2v_ref[...]) + b2v_ref[...]
    sig = jnp.exp(0.5 * lv)
    m0_ref[...] = m0
    m1_ref[...] = m1
    lv_ref[...] = lv
    sig_ref[...] = sig
    za_ref[...] = m0 + aeps_ref[...] * sig


def _k4(xt, wacat, an0, an1, wan, ab1, ab1v, w2mu, b2mu, w2v, b2v, aeps0):
    spec_row = pl.BlockSpec((BM, 128), lambda i: (i, 0))
    spec_n = pl.BlockSpec((BM, NOISE), lambda i: (i, 0))
    spec_w = pl.BlockSpec((128, 128), lambda i: (0, 0))
    spec_wn = pl.BlockSpec((NOISE, 128), lambda i: (0, 0))
    spec_b = pl.BlockSpec((1, 128), lambda i: (0, 0))
    return pl.pallas_call(
        _k4_body,
        grid=(DBLK,),
        in_specs=[
            pl.BlockSpec((BM, N), lambda i: (i, 0)),
            pl.BlockSpec((N, 2 * HID), lambda i: (0, 0)),
            spec_n, spec_n, spec_wn, spec_b, spec_b,
            spec_w, spec_b, spec_w, spec_b, spec_row,
        ],
        out_specs=[spec_row] * 5,
        out_shape=[jax.ShapeDtypeStruct((D, HID), F32)] * 5,
    )(xt, wacat, an0, an1, wan, ab1, ab1v, w2mu, b2mu, w2v, b2v, aeps0)


def _k5_body(zl_ref, zr_ref, o_ref):
    o_ref[...] = _dot1(zl_ref[...], zr_ref[...])


def _k5(z_u):
    return pl.pallas_call(
        _k5_body,
        grid=(NBLK,),
        in_specs=[
            pl.BlockSpec((BM, HID), lambda i: (i, 0)),
            pl.BlockSpec((N, HID), lambda i: (0, 0)),
        ],
        out_specs=pl.BlockSpec((BM, N), lambda i: (i, 0)),
        out_shape=jax.ShapeDtypeStruct((N, N), F32),
    )(z_u, z_u)


def _k6_body(xt_ref, za_ref, ones_ref, o_ref):
    xt = xt_ref[...]
    xz = _dot0(xt, za_ref[...])
    rs = _dot0(jnp.abs(xt), ones_ref[...])
    o_ref[...] = xz / jnp.maximum(rs, 1e-12)


def _k6(xt, za, ones_d):
    return pl.pallas_call(
        _k6_body,
        grid=(NBLK,),
        in_specs=[
            pl.BlockSpec((D, BM), lambda i: (0, i)),
            pl.BlockSpec((D, HID), lambda i: (0, 0)),
            pl.BlockSpec((D, 128), lambda i: (0, 0)),
        ],
        out_specs=pl.BlockSpec((BM, HID), lambda i: (i, 0)),
        out_shape=jax.ShapeDtypeStruct((N, HID), F32),
    )(xt, za, ones_d)


def _k7_body(cf_ref, wt_ref, a2_ref, h_ref, ee_ref):
    i = pl.program_id(0)
    j = pl.program_id(1)
    h = _dot1(cf_ref[...], wt_ref[...])

    edge = jnp.logical_or(i == NBLK - 1, j == HJ - 1)

    @pl.when(jnp.logical_not(edge))
    def _():
        h_ref[...] = h

    @pl.when(edge)
    def _():
        row = i * BM + jax.lax.broadcasted_iota(jnp.int32, h.shape, 0)
        col = j * HBN + jax.lax.broadcasted_iota(jnp.int32, h.shape, 1)
        h_ref[...] = jnp.where((row < N) & (col < D), h, 0.0)

    @pl.when(j == 0)
    def _():
        ee_ref[...] = jnp.zeros_like(ee_ref)

    ee_ref[...] += _dot(h_ref[...], a2_ref[...])


def _k7(cf, wt, a2):
    return pl.pallas_call(
        _k7_body,
        grid=(NBLK, HJ),
        in_specs=[
            pl.BlockSpec((BM, 2 * HID), lambda i, j: (i, 0)),
            pl.BlockSpec((HBN, 2 * HID), lambda i, j: (j, 0)),
            pl.BlockSpec((HBN, 128), lambda i, j: (j, 0)),
        ],
        out_specs=[
            pl.BlockSpec((BM, HBN), lambda i, j: (i, j)),
            pl.BlockSpec((BM, 128), lambda i, j: (i, 0)),
        ],
        out_shape=[
            jax.ShapeDtypeStruct((NBLK * BM, D), F32),
            jax.ShapeDtypeStruct((NBLK * BM, 128), F32),
        ],
    )(cf, wt, a2)


def _k8_body(ee_ref, elt_ref, adj_ref, h_ref, b_ref, o_ref,
             acc_ref, m_ref, l_ref):
    j = pl.program_id(1)

    @pl.when(j == 0)
    def _():
        acc_ref[...] = jnp.zeros_like(acc_ref)
        m_ref[...] = jnp.full_like(m_ref, -1e30)
        l_ref[...] = jnp.zeros_like(l_ref)

    er = ee_ref[...][:, 1:2]
    el = elt_ref[...]
    e = er + el
    e = jnp.where(e > 0, e, 0.2 * e)
    e = jnp.where(adj_ref[...] > 0, e, -1e9)

    def _masked(ev):
        col = j * GAT_BN + jax.lax.broadcasted_iota(jnp.int32, ev.shape, 1)
        return jnp.where(col < N, ev, -1e9)

    e = jax.lax.cond(j == GAT_NJ - 1, _masked, lambda ev: ev, e)

    m_prev = m_ref[...]
    m_new = jnp.maximum(m_prev, jnp.max(e, axis=1, keepdims=True))
    p = jnp.exp(e - m_new)
    corr = jnp.exp(m_prev - m_new)
    l_ref[...] = l_ref[...] * corr + jnp.sum(p, axis=1, keepdims=True)
    acc_ref[...] = acc_ref[...] * corr + _dot(p, h_ref[...])
    m_ref[...] = m_new

    @pl.when(j == GAT_NJ - 1)
    def _():
        o_ref[...] = jnp.transpose(acc_ref[...] / l_ref[...] + b_ref[...])


def _k8(ee, elt, adj, h, decb):
    return pl.pallas_call(
        _k8_body,
        grid=(NBLK, GAT_NJ),
        in_specs=[
            pl.BlockSpec((GAT_BM, 128), lambda i, j: (i, 0)),
            pl.BlockSpec((1, GAT_BN), lambda i, j: (0, j)),
            pl.BlockSpec((GAT_BM, GAT_BN), lambda i, j: (i, j)),
            pl.BlockSpec((GAT_BN, D), lambda i, j: (j, 0)),
            pl.BlockSpec((1, D), lambda i, j: (0, 0)),
        ],
        out_specs=pl.BlockSpec((D, GAT_BM), lambda i, j: (0, i)),
        out_shape=jax.ShapeDtypeStruct((D, N), F32),
        scratch_shapes=[
            pltpu.VMEM((GAT_BM, D), F32),
            pltpu.VMEM((GAT_BM, 1), F32),
            pltpu.VMEM((GAT_BM, 1), F32),
        ],
    )(ee, elt, adj, h, decb)


def kernel(graph, x, nmu_W1, nmu_b1, nmu_W2, nmu_b2, nvar_W1, nvar_b1,
           nvar_W2, nvar_b2, amu_W1, amu_b1, amu_W2, amu_b2, avar_W1,
           avar_b1, avar_W2, avar_b2, dec_W, dec_al, dec_ar, dec_b):
    f32 = F32
    node_noise, attr_noise, node_eps0, attr_eps0 = _rng_consts()
    nn0 = node_noise[:, 0, :]
    nn1 = node_noise[:, 1, :]
    an0 = attr_noise[:, 0, :]
    an1 = attr_noise[:, 1, :]

    xt = x.T
    wt = dec_W.T

    wcat = jnp.concatenate([nmu_W1[NOISE:], nvar_W1], axis=1)
    wn = nmu_W1[:NOISE]
    b1 = nmu_b1.reshape(1, HID)
    b1v = nvar_b1.reshape(1, HID)

    wbd = jnp.zeros((3 * HID, 3 * HID), f32)
    wbd = wbd.at[:HID, :HID].set(nmu_W2)
    wbd = wbd.at[HID:2 * HID, HID:2 * HID].set(nmu_W2)
    wbd = wbd.at[2 * HID:, 2 * HID:].set(nvar_W2)
    b2cat = jnp.concatenate([nmu_b2, nmu_b2, nvar_b2]).reshape(1, 3 * HID)

    wacat = jnp.concatenate([amu_W1[NOISE:], avar_W1], axis=1)
    wan = amu_W1[:NOISE]

    s1 = _k1(xt, wcat, nn0, nn1, wn, b1, b1v)
    s2 = _k2(graph, s1, wbd, b2cat)
    m_all, z_u, sig_n = _k3(graph, s2, node_eps0)

    am0, am1, alv, asig, z_a = _k4(
        xt, wacat, an0, an1, wan,
        amu_b1.reshape(1, 128), avar_b1.reshape(1, 128),
        amu_W2, amu_b2.reshape(1, 128), avar_W2, avar_b2.reshape(1, 128),
        attr_eps0)

    links = _k5(z_u)
    ones_d = jnp.ones((D, 128), f32)
    fine = _k6(xt, z_a, ones_d)
    cf = jnp.concatenate([z_u, fine], axis=1)
    a2 = jnp.zeros((HJ * HBN, 128), f32)
    a2 = a2.at[:D, 0].set(dec_al)
    a2 = a2.at[:D, 1].set(dec_ar)
    h, ee = _k7(cf, wt, a2)
    elt = ee[:, 0:1].T
    out_at = _k8(ee, elt, graph, h, dec_b.reshape(1, D))

    node_mu0 = m_all[:, :HID]
    node_mu1 = m_all[:, HID:2 * HID]
    node_logv = m_all[:, 2 * HID:]

    merged_node_mu = jnp.stack([node_mu1, node_mu0], axis=1)[:, None, :, :]
    merged_node_sigma = jnp.repeat(sig_n[:, None, None, :], 2, axis=2)
    merged_node_z = jnp.repeat(z_u[:, None, None, :], 2, axis=2)
    node_logv_iw = node_logv[:, None, :]
    node_z_iw = z_u[:, None, :]

    merged_attr_mu = jnp.stack([am1, am0], axis=1)[:, None, :, :]
    merged_attr_sigma = jnp.repeat(asig[:, None, None, :], 2, axis=2)
    merged_attr_z = jnp.repeat(z_a[:, None, None, :], 2, axis=2)
    attr_logv_iw = alv[:, None, :]
    attr_z_iw = z_a[:, None, :]

    reconstruct_node_logits = links[:, :, None]
    reconstruct_attr_logits = out_at.T[:, :, None]

    return (merged_node_mu, merged_node_sigma, merged_node_z, node_logv_iw,
            node_z_iw, merged_attr_mu, merged_attr_sigma, merged_attr_z,
            attr_logv_iw, attr_z_iw, reconstruct_node_logits,
            reconstruct_attr_logits, node_mu0, am0)


# speedup vs baseline: 1.9701x; 1.1632x over previous
import jax
import jax.numpy as jnp
from jax.experimental import pallas as pl
from jax.experimental.pallas import tpu as pltpu

N = 2708
D = 1433
NOISE = 5
HID = 128
OUT = 128
F32 = jnp.float32

BM = 256
NBLK = 11
DBLK = 6
HBN = 512
HJ = 3
GAT_BM = 256
GAT_BN = 1408
GAT_NJ = 2


def _rng_consts():
    rk = jax.random.key(7)
    r = jax.random.split(rk, 4)
    node_noise = jax.random.bernoulli(r[0], 0.5, (N, 2, NOISE)).astype(F32)
    attr_noise = jax.random.bernoulli(r[1], 0.5, (D, 2, NOISE)).astype(F32)
    node_eps0 = jax.random.normal(r[2], (N, 1, OUT), dtype=F32)[:, 0, :]
    attr_eps0 = jax.random.normal(r[3], (D, 1, 128), dtype=F32)[:, 0, :]
    return node_noise, attr_noise, node_eps0, attr_eps0


def _dot(a, b):
    return jnp.dot(a, b, preferred_element_type=F32)


def _dot0(a, b):
    return jax.lax.dot_general(a, b, (((0,), (0,)), ((), ())),
                               preferred_element_type=F32)


def _dot1(a, b):
    return jax.lax.dot_general(a, b, (((1,), (1,)), ((), ())),
                               preferred_element_type=F32)


def _k1_body(xt_ref, w_ref, nn0_ref, nn1_ref, wn_ref, b1_ref, b1v_ref, o_ref):
    acc = _dot0(xt_ref[...], w_ref[...])
    xa = acc[:, :HID] + b1_ref[...]
    g1 = acc[:, HID:] + b1v_ref[...]
    h0 = xa + _dot(nn0_ref[...], wn_ref[...])
    h1 = xa + _dot(nn1_ref[...], wn_ref[...])
    o_ref[...] = jnp.concatenate([h0, h1, g1], axis=1)


def _k1(xt, wcat, nn0, nn1, wn, b1, b1v):
    return pl.pallas_call(
        _k1_body,
        grid=(NBLK,),
        in_specs=[
            pl.BlockSpec((D, BM), lambda i: (0, i)),
            pl.BlockSpec((D, 2 * HID), lambda i: (0, 0)),
            pl.BlockSpec((BM, NOISE), lambda i: (i, 0)),
            pl.BlockSpec((BM, NOISE), lambda i: (i, 0)),
            pl.BlockSpec((NOISE, HID), lambda i: (0, 0)),
            pl.BlockSpec((1, HID), lambda i: (0, 0)),
            pl.BlockSpec((1, HID), lambda i: (0, 0)),
        ],
        out_specs=pl.BlockSpec((BM, 3 * HID), lambda i: (i, 0)),
        out_shape=jax.ShapeDtypeStruct((N, 3 * HID), F32),
    )(xt, wcat, nn0, nn1, wn, b1, b1v)


def _k2_body(adj_ref, s1_ref, wbd_ref, b2_ref, o_ref):
    t = _dot(adj_ref[...], s1_ref[...])
    o_ref[...] = _dot(jnp.maximum(t, 0.0), wbd_ref[...]) + b2_ref[...]


def _k2(adj, s1, wbd, b2cat):
    return pl.pallas_call(
        _k2_body,
        grid=(NBLK,),
        in_specs=[
            pl.BlockSpec((BM, N), lambda i: (i, 0)),
            pl.BlockSpec((N, 3 * HID), lambda i: (0, 0)),
            pl.BlockSpec((3 * HID, 3 * HID), lambda i: (0, 0)),
            pl.BlockSpec((1, 3 * HID), lambda i: (0, 0)),
        ],
        out_specs=pl.BlockSpec((BM, 3 * HID), lambda i: (i, 0)),
        out_shape=jax.ShapeDtypeStruct((N, 3 * HID), F32),
    )(adj, s1, wbd, b2cat)


def _k3_body(adj_ref, s2_ref, eps_ref, m_ref, z_ref, sig_ref):
    m = _dot(adj_ref[...], s2_ref[...])
    m_ref[...] = m
    sig = jnp.exp(0.5 * m[:, 2 * HID:])
    sig_ref[...] = sig
    z_ref[...] = m[:, :HID] + eps_ref[...] * sig


def _k3(adj, s2, eps0):
    return pl.pallas_call(
        _k3_body,
        grid=(NBLK,),
        in_specs=[
            pl.BlockSpec((BM, N), lambda i: (i, 0)),
            pl.BlockSpec((N, 3 * HID), lambda i: (0, 0)),
            pl.BlockSpec((BM, HID), lambda i: (i, 0)),
        ],
        out_specs=[
            pl.BlockSpec((BM, 3 * HID), lambda i: (i, 0)),
            pl.BlockSpec((BM, HID), lambda i: (i, 0)),
            pl.BlockSpec((BM, HID), lambda i: (i, 0)),
        ],
        out_shape=[
            jax.ShapeDtypeStruct((N, 3 * HID), F32),
            jax.ShapeDtypeStruct((N, HID), F32),
            jax.ShapeDtypeStruct((N, HID), F32),
        ],
    )(adj, s2, eps0)


def _k4_body(xt_ref, w_ref, an0_ref, an1_ref, wan_ref, b1_ref, b1v_ref,
             w2mu_ref, b2mu_ref, w2v_ref, b2v_ref, aeps_ref,
             m0_ref, m1_ref, lv_ref, sig_ref, za_ref):
    a = _dot(xt_ref[...], w_ref[...])
    base = a[:, :HID] + b1_ref[...]
    n0 = _dot(an0_ref[...], wan_ref[...])
    n1 = _dot(an1_ref[...], wan_ref[...])
    u0 = jnp.maximum(base + n0, 0.0)
    u1 = jnp.maximum(base + n1, 0.0)
    v = jnp.maximum(a[:, HID:] + b1v_ref[...], 0.0)
    m0 = _dot(u0, w2mu_ref[...]) + b2mu_ref[...]
    m1 = _dot(u1, w2mu_ref[...]) + b2mu_ref[...]
    lv = _dot(v, w2v_ref[...]) + b2v_ref[...]
    sig = jnp.exp(0.5 * lv)
    m0_ref[...] = m0
    m1_ref[...] = m1
    lv_ref[...] = lv
    sig_ref[...] = sig
    za_ref[...] = m0 + aeps_ref[...] * sig


def _k4(xt, wacat, an0, an1, wan, ab1, ab1v, w2mu, b2mu, w2v, b2v, aeps0):
    spec_row = pl.BlockSpec((BM, 128), lambda i: (i, 0))
    spec_n = pl.BlockSpec((BM, NOISE), lambda i: (i, 0))
    spec_w = pl.BlockSpec((128, 128), lambda i: (0, 0))
    spec_wn = pl.BlockSpec((NOISE, 128), lambda i: (0, 0))
    spec_b = pl.BlockSpec((1, 128), lambda i: (0, 0))
    return pl.pallas_call(
        _k4_body,
        grid=(DBLK,),
        in_specs=[
            pl.BlockSpec((BM, N), lambda i: (i, 0)),
            pl.BlockSpec((N, 2 * HID), lambda i: (0, 0)),
            spec_n, spec_n, spec_wn, spec_b, spec_b,
            spec_w, spec_b, spec_w, spec_b, spec_row,
        ],
        out_specs=[spec_row] * 5,
        out_shape=[jax.ShapeDtypeStruct((D, HID), F32)] * 5,
    )(xt, wacat, an0, an1, wan, ab1, ab1v, w2mu, b2mu, w2v, b2v, aeps0)


def _k567_body(zb_ref, zall_ref, xt_ref, za_ref, ones_ref, wt_ref, a2_ref,
               links_ref, h_ref, ee_ref):
    zb = zb_ref[...]
    links_ref[...] = _dot1(zb, zall_ref[...])
    xt = xt_ref[...]
    xz = _dot0(xt, za_ref[...])
    rs = _dot0(jnp.abs(xt), ones_ref[...])
    fine = xz / jnp.maximum(rs, 1e-12)
    wt = wt_ref[...]
    h = _dot1(zb, wt[:, :HID]) + _dot1(fine, wt[:, HID:])
    h_ref[...] = h
    ee_ref[...] = _dot(h, a2_ref[...])


def _k567(z_u, xt, za, ones_d, wt, a2):
    return pl.pallas_call(
        _k567_body,
        grid=(NBLK,),
        in_specs=[
            pl.BlockSpec((BM, HID), lambda i: (i, 0)),
            pl.BlockSpec((N, HID), lambda i: (0, 0)),
            pl.BlockSpec((D, BM), lambda i: (0, i)),
            pl.BlockSpec((D, HID), lambda i: (0, 0)),
            pl.BlockSpec((D, 128), lambda i: (0, 0)),
            pl.BlockSpec((D, 2 * HID), lambda i: (0, 0)),
            pl.BlockSpec((D, 128), lambda i: (0, 0)),
        ],
        out_specs=[
            pl.BlockSpec((BM, N), lambda i: (i, 0)),
            pl.BlockSpec((BM, D), lambda i: (i, 0)),
            pl.BlockSpec((BM, 128), lambda i: (i, 0)),
        ],
        out_shape=[
            jax.ShapeDtypeStruct((N, N), F32),
            jax.ShapeDtypeStruct((N, D), F32),
            jax.ShapeDtypeStruct((N, 128), F32),
        ],
    )(z_u, z_u, xt, za, ones_d, wt, a2)


def _k8_body(ee_ref, elt_ref, adj_ref, h_ref, b_ref, o_ref):
    er = ee_ref[...][:, 1:2]
    el = elt_ref[...]
    e = er + el
    e = jnp.where(e > 0, e, 0.2 * e)
    e = jnp.where(adj_ref[...] > 0, e, -1e9)
    m = jnp.max(e, axis=1, keepdims=True)
    p = jnp.exp(e - m)
    l = jnp.sum(p, axis=1, keepdims=True)
    out = _dot(p, h_ref[...]) / l + b_ref[...]
    o_ref[...] = jnp.transpose(out)


def _k8(ee, elt, adj, h, decb):
    return pl.pallas_call(
        _k8_body,
        grid=(NBLK,),
        in_specs=[
            pl.BlockSpec((GAT_BM, 128), lambda i: (i, 0)),
            pl.BlockSpec((1, N), lambda i: (0, 0)),
            pl.BlockSpec((GAT_BM, N), lambda i: (i, 0)),
            pl.BlockSpec((N, D), lambda i: (0, 0)),
            pl.BlockSpec((1, D), lambda i: (0, 0)),
        ],
        out_specs=pl.BlockSpec((D, GAT_BM), lambda i: (0, i)),
        out_shape=jax.ShapeDtypeStruct((D, N), F32),
    )(ee, elt, adj, h, decb)


def kernel(graph, x, nmu_W1, nmu_b1, nmu_W2, nmu_b2, nvar_W1, nvar_b1,
           nvar_W2, nvar_b2, amu_W1, amu_b1, amu_W2, amu_b2, avar_W1,
           avar_b1, avar_W2, avar_b2, dec_W, dec_al, dec_ar, dec_b):
    f32 = F32
    node_noise, attr_noise, node_eps0, attr_eps0 = _rng_consts()
    nn0 = node_noise[:, 0, :]
    nn1 = node_noise[:, 1, :]
    an0 = attr_noise[:, 0, :]
    an1 = attr_noise[:, 1, :]

    xt = x.T
    wt = dec_W.T

    wcat = jnp.concatenate([nmu_W1[NOISE:], nvar_W1], axis=1)
    wn = nmu_W1[:NOISE]
    b1 = nmu_b1.reshape(1, HID)
    b1v = nvar_b1.reshape(1, HID)

    wbd = jnp.zeros((3 * HID, 3 * HID), f32)
    wbd = wbd.at[:HID, :HID].set(nmu_W2)
    wbd = wbd.at[HID:2 * HID, HID:2 * HID].set(nmu_W2)
    wbd = wbd.at[2 * HID:, 2 * HID:].set(nvar_W2)
    b2cat = jnp.concatenate([nmu_b2, nmu_b2, nvar_b2]).reshape(1, 3 * HID)

    wacat = jnp.concatenate([amu_W1[NOISE:], avar_W1], axis=1)
    wan = amu_W1[:NOISE]

    s1 = _k1(xt, wcat, nn0, nn1, wn, b1, b1v)
    s2 = _k2(graph, s1, wbd, b2cat)
    m_all, z_u, sig_n = _k3(graph, s2, node_eps0)

    am0, am1, alv, asig, z_a = _k4(
        xt, wacat, an0, an1, wan,
        amu_b1.reshape(1, 128), avar_b1.reshape(1, 128),
        amu_W2, amu_b2.reshape(1, 128), avar_W2, avar_b2.reshape(1, 128),
        attr_eps0)

    ones_d = jnp.ones((D, 128), f32)
    a2 = jnp.zeros((D, 128), f32)
    a2 = a2.at[:, 0].set(dec_al)
    a2 = a2.at[:, 1].set(dec_ar)
    links, h, ee = _k567(z_u, xt, z_a, ones_d, wt, a2)
    elt = ee[:, 0:1].T
    out_at = _k8(ee, elt, graph, h, dec_b.reshape(1, D))

    node_mu0 = m_all[:, :HID]
    node_mu1 = m_all[:, HID:2 * HID]
    node_logv = m_all[:, 2 * HID:]

    merged_node_mu = jnp.stack([node_mu1, node_mu0], axis=1)[:, None, :, :]
    merged_node_sigma = jnp.repeat(sig_n[:, None, None, :], 2, axis=2)
    merged_node_z = jnp.repeat(z_u[:, None, None, :], 2, axis=2)
    node_logv_iw = node_logv[:, None, :]
    node_z_iw = z_u[:, None, :]

    merged_attr_mu = jnp.stack([am1, am0], axis=1)[:, None, :, :]
    merged_attr_sigma = jnp.repeat(asig[:, None, None, :], 2, axis=2)
    merged_attr_z = jnp.repeat(z_a[:, None, None, :], 2, axis=2)
    attr_logv_iw = alv[:, None, :]
    attr_z_iw = z_a[:, None, :]

    reconstruct_node_logits = links[:, :, None]
    reconstruct_attr_logits = out_at.T[:, :, None]

    return (merged_node_mu, merged_node_sigma, merged_node_z, node_logv_iw,
            node_z_iw, merged_attr_mu, merged_attr_sigma, merged_attr_z,
            attr_logv_iw, attr_z_iw, reconstruct_node_logits,
            reconstruct_attr_logits, node_mu0, am0)


# speedup vs baseline: 2.0128x; 1.0217x over previous
import jax
import jax.numpy as jnp
from jax.experimental import pallas as pl
from jax.experimental.pallas import tpu as pltpu

N = 2708
D = 1433
NOISE = 5
HID = 128
OUT = 128
F32 = jnp.float32

BM = 256
NBLK = 11
DBLK = 6
HBN = 512
HJ = 3
GAT_BM = 256
GAT_BN = 1408
GAT_NJ = 2


def _rng_consts():
    rk = jax.random.key(7)
    r = jax.random.split(rk, 4)
    node_noise = jax.random.bernoulli(r[0], 0.5, (N, 2, NOISE)).astype(F32)
    attr_noise = jax.random.bernoulli(r[1], 0.5, (D, 2, NOISE)).astype(F32)
    node_eps0 = jax.random.normal(r[2], (N, 1, OUT), dtype=F32)[:, 0, :]
    attr_eps0 = jax.random.normal(r[3], (D, 1, 128), dtype=F32)[:, 0, :]
    return node_noise, attr_noise, node_eps0, attr_eps0


def _dot(a, b):
    return jnp.dot(a, b, preferred_element_type=F32)


def _dot0(a, b):
    return jax.lax.dot_general(a, b, (((0,), (0,)), ((), ())),
                               preferred_element_type=F32)


def _dot1(a, b):
    return jax.lax.dot_general(a, b, (((1,), (1,)), ((), ())),
                               preferred_element_type=F32)


def _k1_body(xt_ref, w_ref, nn0_ref, nn1_ref, wn_ref, b1_ref, b1v_ref, o_ref):
    acc = _dot0(xt_ref[...], w_ref[...])
    xa = acc[:, :HID] + b1_ref[...]
    g1 = acc[:, HID:] + b1v_ref[...]
    h0 = xa + _dot(nn0_ref[...], wn_ref[...])
    h1 = xa + _dot(nn1_ref[...], wn_ref[...])
    o_ref[...] = jnp.concatenate([h0, h1, g1], axis=1)


def _k1(xt, wcat, nn0, nn1, wn, b1, b1v):
    return pl.pallas_call(
        _k1_body,
        grid=(NBLK,),
        in_specs=[
            pl.BlockSpec((D, BM), lambda i: (0, i)),
            pl.BlockSpec((D, 2 * HID), lambda i: (0, 0)),
            pl.BlockSpec((BM, NOISE), lambda i: (i, 0)),
            pl.BlockSpec((BM, NOISE), lambda i: (i, 0)),
            pl.BlockSpec((NOISE, HID), lambda i: (0, 0)),
            pl.BlockSpec((1, HID), lambda i: (0, 0)),
            pl.BlockSpec((1, HID), lambda i: (0, 0)),
        ],
        out_specs=pl.BlockSpec((BM, 3 * HID), lambda i: (i, 0)),
        out_shape=jax.ShapeDtypeStruct((N, 3 * HID), F32),
    )(xt, wcat, nn0, nn1, wn, b1, b1v)


def _k2_body(adj_ref, s1_ref, wbd_ref, b2_ref, o_ref):
    t = _dot(adj_ref[...], s1_ref[...])
    o_ref[...] = _dot(jnp.maximum(t, 0.0), wbd_ref[...]) + b2_ref[...]


def _k2(adj, s1, wbd, b2cat):
    return pl.pallas_call(
        _k2_body,
        grid=(NBLK,),
        in_specs=[
            pl.BlockSpec((BM, N), lambda i: (i, 0)),
            pl.BlockSpec((N, 3 * HID), lambda i: (0, 0)),
            pl.BlockSpec((3 * HID, 3 * HID), lambda i: (0, 0)),
            pl.BlockSpec((1, 3 * HID), lambda i: (0, 0)),
        ],
        out_specs=pl.BlockSpec((BM, 3 * HID), lambda i: (i, 0)),
        out_shape=jax.ShapeDtypeStruct((N, 3 * HID), F32),
    )(adj, s1, wbd, b2cat)


def _k3_body(adj_ref, s2_ref, eps_ref, m_ref, z_ref, sig_ref):
    m = _dot(adj_ref[...], s2_ref[...])
    m_ref[...] = m
    sig = jnp.exp(0.5 * m[:, 2 * HID:])
    sig_ref[...] = sig
    z_ref[...] = m[:, :HID] + eps_ref[...] * sig


def _k3(adj, s2, eps0):
    return pl.pallas_call(
        _k3_body,
        grid=(NBLK,),
        in_specs=[
            pl.BlockSpec((BM, N), lambda i: (i, 0)),
            pl.BlockSpec((N, 3 * HID), lambda i: (0, 0)),
            pl.BlockSpec((BM, HID), lambda i: (i, 0)),
        ],
        out_specs=[
            pl.BlockSpec((BM, 3 * HID), lambda i: (i, 0)),
            pl.BlockSpec((BM, HID), lambda i: (i, 0)),
            pl.BlockSpec((BM, HID), lambda i: (i, 0)),
        ],
        out_shape=[
            jax.ShapeDtypeStruct((N, 3 * HID), F32),
            jax.ShapeDtypeStruct((N, HID), F32),
            jax.ShapeDtypeStruct((N, HID), F32),
        ],
    )(adj, s2, eps0)


def _k4_body(xt_ref, w_ref, an0_ref, an1_ref, wan_ref, b1_ref, b1v_ref,
             w2mu_ref, b2mu_ref, w2v_ref, b2v_ref, aeps_ref,
             m0_ref, m1_ref, lv_ref, sig_ref, za_ref):
    a = _dot(xt_ref[...], w_ref[...])
    base = a[:, :HID] + b1_ref[...]
    n0 = _dot(an0_ref[...], wan_ref[...])
    n1 = _dot(an1_ref[...], wan_ref[...])
    u0 = jnp.maximum(base + n0, 0.0)
    u1 = jnp.maximum(base + n1, 0.0)
    v = jnp.maximum(a[:, HID:] + b1v_ref[...], 0.0)
    m0 = _dot(u0, w2mu_ref[...]) + b2mu_ref[...]
    m1 = _dot(u1, w2mu_ref[...]) + b2mu_ref[...]
    lv = _dot(v, w2v_ref[...]) + b2v_ref[...]
    sig = jnp.exp(0.5 * lv)
    m0_ref[...] = m0
    m1_ref[...] = m1
    lv_ref[...] = lv
    sig_ref[...] = sig
    za_ref[...] = m0 + aeps_ref[...] * sig


def _k4(xt, wacat, an0, an1, wan, ab1, ab1v, w2mu, b2mu, w2v, b2v, aeps0):
    spec_row = pl.BlockSpec((BM, 128), lambda i: (i, 0))
    spec_n = pl.BlockSpec((BM, NOISE), lambda i: (i, 0))
    spec_w = pl.BlockSpec((128, 128), lambda i: (0, 0))
    spec_wn = pl.BlockSpec((NOISE, 128), lambda i: (0, 0))
    spec_b = pl.BlockSpec((1, 128), lambda i: (0, 0))
    return pl.pallas_call(
        _k4_body,
        grid=(DBLK,),
        in_specs=[
            pl.BlockSpec((BM, N), lambda i: (i, 0)),
            pl.BlockSpec((N, 2 * HID), lambda i: (0, 0)),
            spec_n, spec_n, spec_wn, spec_b, spec_b,
            spec_w, spec_b, spec_w, spec_b, spec_row,
        ],
        out_specs=[spec_row] * 5,
        out_shape=[jax.ShapeDtypeStruct((D, HID), F32)] * 5,
    )(xt, wacat, an0, an1, wan, ab1, ab1v, w2mu, b2mu, w2v, b2v, aeps0)


def _k5_body(zb_ref, zall_ref, o_ref):
    o_ref[...] = _dot1(zb_ref[...], zall_ref[...])


def _k5(z_u):
    return pl.pallas_call(
        _k5_body,
        grid=(NBLK,),
        in_specs=[
            pl.BlockSpec((BM, HID), lambda i: (i, 0)),
            pl.BlockSpec((N, HID), lambda i: (0, 0)),
        ],
        out_specs=pl.BlockSpec((BM, N), lambda i: (i, 0)),
        out_shape=jax.ShapeDtypeStruct((N, N), F32),
    )(z_u, z_u)


def _k67_body(zb_ref, xt_ref, za_ref, ones_ref, wt_ref, a2_ref,
              h_ref, ee_ref):
    zb = zb_ref[...]
    xt = xt_ref[...]
    xz = _dot0(xt, za_ref[...])
    rs = _dot0(jnp.abs(xt), ones_ref[...])
    fine = xz / jnp.maximum(rs, 1e-12)
    wt = wt_ref[...]
    h = _dot1(zb, wt[:, :HID]) + _dot1(fine, wt[:, HID:])
    h_ref[...] = h.astype(jnp.bfloat16)
    ee_ref[...] = _dot(h, a2_ref[...])


def _k67(z_u, xt, za, ones_d, wt, a2):
    return pl.pallas_call(
        _k67_body,
        grid=(NBLK,),
        in_specs=[
            pl.BlockSpec((BM, HID), lambda i: (i, 0)),
            pl.BlockSpec((D, BM), lambda i: (0, i)),
            pl.BlockSpec((D, HID), lambda i: (0, 0)),
            pl.BlockSpec((D, 128), lambda i: (0, 0)),
            pl.BlockSpec((D, 2 * HID), lambda i: (0, 0)),
            pl.BlockSpec((D, 128), lambda i: (0, 0)),
        ],
        out_specs=[
            pl.BlockSpec((BM, D), lambda i: (i, 0)),
            pl.BlockSpec((BM, 128), lambda i: (i, 0)),
        ],
        out_shape=[
            jax.ShapeDtypeStruct((N, D), jnp.bfloat16),
            jax.ShapeDtypeStruct((N, 128), F32),
        ],
    )(z_u, xt, za, ones_d, wt, a2)


def _k8_body(ee_ref, elt_ref, adj_ref, h_ref, b_ref, o_ref):
    er = ee_ref[...][:, 1:2]
    el = elt_ref[...]
    e = er + el
    e = jnp.maximum(e, 0.2 * e)
    e = jnp.where(adj_ref[...] > 0, e, -1e9)
    m = jnp.max(e, axis=1, keepdims=True)
    p = jnp.exp(e - m)
    l = jnp.sum(p, axis=1, keepdims=True)
    out = _dot(p.astype(jnp.bfloat16), h_ref[...]) / l + b_ref[...]
    o_ref[...] = jnp.transpose(out)


def _k8(ee, elt, adj, h, decb):
    return pl.pallas_call(
        _k8_body,
        grid=(NBLK,),
        in_specs=[
            pl.BlockSpec((GAT_BM, 128), lambda i: (i, 0)),
            pl.BlockSpec((1, N), lambda i: (0, 0)),
            pl.BlockSpec((GAT_BM, N), lambda i: (i, 0)),
            pl.BlockSpec((N, D), lambda i: (0, 0)),
            pl.BlockSpec((1, D), lambda i: (0, 0)),
        ],
        out_specs=pl.BlockSpec((D, GAT_BM), lambda i: (0, i)),
        out_shape=jax.ShapeDtypeStruct((D, N), F32),
    )(ee, elt, adj, h, decb)


def kernel(graph, x, nmu_W1, nmu_b1, nmu_W2, nmu_b2, nvar_W1, nvar_b1,
           nvar_W2, nvar_b2, amu_W1, amu_b1, amu_W2, amu_b2, avar_W1,
           avar_b1, avar_W2, avar_b2, dec_W, dec_al, dec_ar, dec_b):
    f32 = F32
    node_noise, attr_noise, node_eps0, attr_eps0 = _rng_consts()
    nn0 = node_noise[:, 0, :]
    nn1 = node_noise[:, 1, :]
    an0 = attr_noise[:, 0, :]
    an1 = attr_noise[:, 1, :]

    xt = x.T
    wt = dec_W.T

    wcat = jnp.concatenate([nmu_W1[NOISE:], nvar_W1], axis=1)
    wn = nmu_W1[:NOISE]
    b1 = nmu_b1.reshape(1, HID)
    b1v = nvar_b1.reshape(1, HID)

    wbd = jnp.zeros((3 * HID, 3 * HID), f32)
    wbd = wbd.at[:HID, :HID].set(nmu_W2)
    wbd = wbd.at[HID:2 * HID, HID:2 * HID].set(nmu_W2)
    wbd = wbd.at[2 * HID:, 2 * HID:].set(nvar_W2)
    b2cat = jnp.concatenate([nmu_b2, nmu_b2, nvar_b2]).reshape(1, 3 * HID)

    wacat = jnp.concatenate([amu_W1[NOISE:], avar_W1], axis=1)
    wan = amu_W1[:NOISE]

    s1 = _k1(xt, wcat, nn0, nn1, wn, b1, b1v)
    s2 = _k2(graph, s1, wbd, b2cat)
    m_all, z_u, sig_n = _k3(graph, s2, node_eps0)

    links = _k5(z_u)

    am0, am1, alv, asig, z_a = _k4(
        xt, wacat, an0, an1, wan,
        amu_b1.reshape(1, 128), avar_b1.reshape(1, 128),
        amu_W2, amu_b2.reshape(1, 128), avar_W2, avar_b2.reshape(1, 128),
        attr_eps0)

    ones_d = jnp.ones((D, 128), f32)
    a2 = jnp.zeros((D, 128), f32)
    a2 = a2.at[:, 0].set(dec_al)
    a2 = a2.at[:, 1].set(dec_ar)
    h, ee = _k67(z_u, xt, z_a, ones_d, wt, a2)
    elt = ee[:, 0:1].T
    out_at = _k8(ee, elt, graph, h, dec_b.reshape(1, D))

    node_mu0 = m_all[:, :HID]
    node_mu1 = m_all[:, HID:2 * HID]
    node_logv = m_all[:, 2 * HID:]

    merged_node_mu = jnp.stack([node_mu1, node_mu0], axis=1)[:, None, :, :]
    merged_node_sigma = jnp.repeat(sig_n[:, None, None, :], 2, axis=2)
    merged_node_z = jnp.repeat(z_u[:, None, None, :], 2, axis=2)
    node_logv_iw = node_logv[:, None, :]
    node_z_iw = z_u[:, None, :]

    merged_attr_mu = jnp.stack([am1, am0], axis=1)[:, None, :, :]
    merged_attr_sigma = jnp.repeat(asig[:, None, None, :], 2, axis=2)
    merged_attr_z = jnp.repeat(z_a[:, None, None, :], 2, axis=2)
    attr_logv_iw = alv[:, None, :]
    attr_z_iw = z_a[:, None, :]

    reconstruct_node_logits = links[:, :, None]
    reconstruct_attr_logits = out_at.T[:, :, None]

    return (merged_node_mu, merged_node_sigma, merged_node_z, node_logv_iw,
            node_z_iw, merged_attr_mu, merged_attr_sigma, merged_attr_z,
            attr_logv_iw, attr_z_iw, reconstruct_node_logits,
            reconstruct_attr_logits, node_mu0, am0)


# speedup vs baseline: 2.3333x; 1.1592x over previous
import jax
import jax.numpy as jnp
from jax.experimental import pallas as pl
from jax.experimental.pallas import tpu as pltpu

N = 2708
D = 1433
NOISE = 5
HID = 128
OUT = 128
F32 = jnp.float32

BM = 256
NBLK = 11
DBLK = 6
HBN = 512
HJ = 3
GAT_BM = 256
GAT_BN = 1408
GAT_NJ = 2


def _rng_consts():
    rk = jax.random.key(7)
    r = jax.random.split(rk, 4)
    node_noise = jax.random.bernoulli(r[0], 0.5, (N, 2, NOISE)).astype(F32)
    attr_noise = jax.random.bernoulli(r[1], 0.5, (D, 2, NOISE)).astype(F32)
    node_eps0 = jax.random.normal(r[2], (N, 1, OUT), dtype=F32)[:, 0, :]
    attr_eps0 = jax.random.normal(r[3], (D, 1, 128), dtype=F32)[:, 0, :]
    return node_noise, attr_noise, node_eps0, attr_eps0


try:
    _RNG_CONSTS = tuple(jax.device_get(t) for t in _rng_consts())
except Exception:
    _RNG_CONSTS = None


def _get_rng_consts():
    if _RNG_CONSTS is not None:
        return tuple(jnp.asarray(t) for t in _RNG_CONSTS)
    return _rng_consts()


def _dot(a, b):
    return jnp.dot(a, b, preferred_element_type=F32)


def _dot0(a, b):
    return jax.lax.dot_general(a, b, (((0,), (0,)), ((), ())),
                               preferred_element_type=F32)


def _dot1(a, b):
    return jax.lax.dot_general(a, b, (((1,), (1,)), ((), ())),
                               preferred_element_type=F32)


def _k1_body(xt_ref, w_ref, nn0_ref, nn1_ref, wn_ref, b1_ref, b1v_ref, o_ref):
    acc = _dot0(xt_ref[...], w_ref[...])
    xa = acc[:, :HID] + b1_ref[...]
    g1 = acc[:, HID:] + b1v_ref[...]
    h0 = xa + _dot(nn0_ref[...], wn_ref[...])
    h1 = xa + _dot(nn1_ref[...], wn_ref[...])
    o_ref[...] = jnp.concatenate([h0, h1, g1], axis=1)


def _k1(xt, wcat, nn0, nn1, wn, b1, b1v):
    return pl.pallas_call(
        _k1_body,
        grid=(NBLK,),
        in_specs=[
            pl.BlockSpec((D, BM), lambda i: (0, i)),
            pl.BlockSpec((D, 2 * HID), lambda i: (0, 0)),
            pl.BlockSpec((BM, NOISE), lambda i: (i, 0)),
            pl.BlockSpec((BM, NOISE), lambda i: (i, 0)),
            pl.BlockSpec((NOISE, HID), lambda i: (0, 0)),
            pl.BlockSpec((1, HID), lambda i: (0, 0)),
            pl.BlockSpec((1, HID), lambda i: (0, 0)),
        ],
        out_specs=pl.BlockSpec((BM, 3 * HID), lambda i: (i, 0)),
        out_shape=jax.ShapeDtypeStruct((N, 3 * HID), F32),
    )(xt, wcat, nn0, nn1, wn, b1, b1v)


def _k2_body(adj_ref, s1_ref, w2mu_ref, b2mu_ref, w2v_ref, b2v_ref, o_ref):
    t = _dot(adj_ref[...], s1_ref[...])
    r = jnp.maximum(t, 0.0)
    o_ref[...] = jnp.concatenate([
        _dot(r[:, :HID], w2mu_ref[...]) + b2mu_ref[...],
        _dot(r[:, HID:2 * HID], w2mu_ref[...]) + b2mu_ref[...],
        _dot(r[:, 2 * HID:], w2v_ref[...]) + b2v_ref[...],
    ], axis=1)


def _k2(adj, s1, w2mu, b2mu, w2v, b2v):
    return pl.pallas_call(
        _k2_body,
        grid=(NBLK,),
        in_specs=[
            pl.BlockSpec((BM, N), lambda i: (i, 0)),
            pl.BlockSpec((N, 3 * HID), lambda i: (0, 0)),
            pl.BlockSpec((HID, HID), lambda i: (0, 0)),
            pl.BlockSpec((1, HID), lambda i: (0, 0)),
            pl.BlockSpec((HID, HID), lambda i: (0, 0)),
            pl.BlockSpec((1, HID), lambda i: (0, 0)),
        ],
        out_specs=pl.BlockSpec((BM, 3 * HID), lambda i: (i, 0)),
        out_shape=jax.ShapeDtypeStruct((N, 3 * HID), F32),
    )(adj, s1, w2mu, b2mu, w2v, b2v)


def _k3_body(adj_ref, s2_ref, eps_ref, m_ref, z_ref, sig_ref):
    m = _dot(adj_ref[...], s2_ref[...])
    m_ref[...] = m
    sig = jnp.exp(0.5 * m[:, 2 * HID:])
    sig_ref[...] = sig
    z_ref[...] = m[:, :HID] + eps_ref[...] * sig


def _k3(adj, s2, eps0):
    return pl.pallas_call(
        _k3_body,
        grid=(NBLK,),
        in_specs=[
            pl.BlockSpec((BM, N), lambda i: (i, 0)),
            pl.BlockSpec((N, 3 * HID), lambda i: (0, 0)),
            pl.BlockSpec((BM, HID), lambda i: (i, 0)),
        ],
        out_specs=[
            pl.BlockSpec((BM, 3 * HID), lambda i: (i, 0)),
            pl.BlockSpec((BM, HID), lambda i: (i, 0)),
            pl.BlockSpec((BM, HID), lambda i: (i, 0)),
        ],
        out_shape=[
            jax.ShapeDtypeStruct((N, 3 * HID), F32),
            jax.ShapeDtypeStruct((N, HID), F32),
            jax.ShapeDtypeStruct((N, HID), F32),
        ],
    )(adj, s2, eps0)


def _k4_body(xt_ref, w_ref, an0_ref, an1_ref, wan_ref, b1_ref, b1v_ref,
             w2mu_ref, b2mu_ref, w2v_ref, b2v_ref, aeps_ref,
             m0_ref, m1_ref, lv_ref, sig_ref, za_ref):
    a = _dot(xt_ref[...], w_ref[...])
    base = a[:, :HID] + b1_ref[...]
    n0 = _dot(an0_ref[...], wan_ref[...])
    n1 = _dot(an1_ref[...], wan_ref[...])
    u0 = jnp.maximum(base + n0, 0.0)
    u1 = jnp.maximum(base + n1, 0.0)
    v = jnp.maximum(a[:, HID:] + b1v_ref[...], 0.0)
    m0 = _dot(u0, w2mu_ref[...]) + b2mu_ref[...]
    m1 = _dot(u1, w2mu_ref[...]) + b2mu_ref[...]
    lv = _dot(v, w2v_ref[...]) + b2v_ref[...]
    sig = jnp.exp(0.5 * lv)
    m0_ref[...] = m0
    m1_ref[...] = m1
    lv_ref[...] = lv
    sig_ref[...] = sig
    za_ref[...] = m0 + aeps_ref[...] * sig


def _k4(xt, wacat, an0, an1, wan, ab1, ab1v, w2mu, b2mu, w2v, b2v, aeps0):
    spec_row = pl.BlockSpec((BM, 128), lambda i: (i, 0))
    spec_n = pl.BlockSpec((BM, NOISE), lambda i: (i, 0))
    spec_w = pl.BlockSpec((128, 128), lambda i: (0, 0))
    spec_wn = pl.BlockSpec((NOISE, 128), lambda i: (0, 0))
    spec_b = pl.BlockSpec((1, 128), lambda i: (0, 0))
    return pl.pallas_call(
        _k4_body,
        grid=(DBLK,),
        in_specs=[
            pl.BlockSpec((BM, N), lambda i: (i, 0)),
            pl.BlockSpec((N, 2 * HID), lambda i: (0, 0)),
            spec_n, spec_n, spec_wn, spec_b, spec_b,
            spec_w, spec_b, spec_w, spec_b, spec_row,
        ],
        out_specs=[spec_row] * 5,
        out_shape=[jax.ShapeDtypeStruct((D, HID), F32)] * 5,
    )(xt, wacat, an0, an1, wan, ab1, ab1v, w2mu, b2mu, w2v, b2v, aeps0)


def _k5_body(zb_ref, zall_ref, o_ref):
    o_ref[...] = _dot1(zb_ref[...], zall_ref[...])


def _k5(z_u):
    return pl.pallas_call(
        _k5_body,
        grid=(NBLK,),
        in_specs=[
            pl.BlockSpec((BM, HID), lambda i: (i, 0)),
            pl.BlockSpec((N, HID), lambda i: (0, 0)),
        ],
        out_specs=pl.BlockSpec((BM, N), lambda i: (i, 0)),
        out_shape=jax.ShapeDtypeStruct((N, N), F32),
    )(z_u, z_u)


def _k67_body(zb_ref, xt_ref, za_ref, ones_ref, wt_ref, a2_ref,
              h_ref, ee_ref):
    zb = zb_ref[...]
    xt = xt_ref[...]
    xz = _dot0(xt, za_ref[...])
    rs = _dot0(jnp.abs(xt), ones_ref[...])
    fine = xz / jnp.maximum(rs, 1e-12)
    wt = wt_ref[...]
    h = _dot1(zb, wt[:, :HID]) + _dot1(fine, wt[:, HID:])
    h_ref[...] = h.astype(jnp.bfloat16)
    ee_ref[...] = _dot(h, a2_ref[...])


def _k67(z_u, xt, za, ones_d, wt, a2):
    return pl.pallas_call(
        _k67_body,
        grid=(NBLK,),
        in_specs=[
            pl.BlockSpec((BM, HID), lambda i: (i, 0)),
            pl.BlockSpec((D, BM), lambda i: (0, i)),
            pl.BlockSpec((D, HID), lambda i: (0, 0)),
            pl.BlockSpec((D, 128), lambda i: (0, 0)),
            pl.BlockSpec((D, 2 * HID), lambda i: (0, 0)),
            pl.BlockSpec((D, 128), lambda i: (0, 0)),
        ],
        out_specs=[
            pl.BlockSpec((BM, D), lambda i: (i, 0)),
            pl.BlockSpec((BM, 128), lambda i: (i, 0)),
        ],
        out_shape=[
            jax.ShapeDtypeStruct((N, D), jnp.bfloat16),
            jax.ShapeDtypeStruct((N, 128), F32),
        ],
    )(z_u, xt, za, ones_d, wt, a2)


def _k8_body(ee_ref, elt_ref, adj_ref, h_ref, b_ref, o_ref):
    er = ee_ref[...][:, 1:2]
    el = elt_ref[...]
    e = er + el
    e = jnp.maximum(e, 0.2 * e)
    e = jnp.where(adj_ref[...] > 0, e, -1e9)
    m = jnp.max(e, axis=1, keepdims=True)
    p = jnp.exp(e - m)
    l = jnp.sum(p, axis=1, keepdims=True)
    out = _dot(p.astype(jnp.bfloat16), h_ref[...]) / l + b_ref[...]
    o_ref[...] = jnp.transpose(out)


def _k8(ee, elt, adj, h, decb):
    return pl.pallas_call(
        _k8_body,
        grid=(NBLK,),
        in_specs=[
            pl.BlockSpec((GAT_BM, 128), lambda i: (i, 0)),
            pl.BlockSpec((1, N), lambda i: (0, 0)),
            pl.BlockSpec((GAT_BM, N), lambda i: (i, 0)),
            pl.BlockSpec((N, D), lambda i: (0, 0)),
            pl.BlockSpec((1, D), lambda i: (0, 0)),
        ],
        out_specs=pl.BlockSpec((D, GAT_BM), lambda i: (0, i)),
        out_shape=jax.ShapeDtypeStruct((D, N), F32),
    )(ee, elt, adj, h, decb)


def kernel(graph, x, nmu_W1, nmu_b1, nmu_W2, nmu_b2, nvar_W1, nvar_b1,
           nvar_W2, nvar_b2, amu_W1, amu_b1, amu_W2, amu_b2, avar_W1,
           avar_b1, avar_W2, avar_b2, dec_W, dec_al, dec_ar, dec_b):
    f32 = F32
    node_noise, attr_noise, node_eps0, attr_eps0 = _get_rng_consts()
    nn0 = node_noise[:, 0, :]
    nn1 = node_noise[:, 1, :]
    an0 = attr_noise[:, 0, :]
    an1 = attr_noise[:, 1, :]

    xt = x.T
    wt = dec_W.T

    wcat = jnp.concatenate([nmu_W1[NOISE:], nvar_W1], axis=1)
    wn = nmu_W1[:NOISE]
    b1 = nmu_b1.reshape(1, HID)
    b1v = nvar_b1.reshape(1, HID)

    wacat = jnp.concatenate([amu_W1[NOISE:], avar_W1], axis=1)
    wan = amu_W1[:NOISE]

    s1 = _k1(xt, wcat, nn0, nn1, wn, b1, b1v)
    s2 = _k2(graph, s1, nmu_W2, nmu_b2.reshape(1, HID),
             nvar_W2, nvar_b2.reshape(1, HID))
    m_all, z_u, sig_n = _k3(graph, s2, node_eps0)

    links = _k5(z_u)

    am0, am1, alv, asig, z_a = _k4(
        xt, wacat, an0, an1, wan,
        amu_b1.reshape(1, 128), avar_b1.reshape(1, 128),
        amu_W2, amu_b2.reshape(1, 128), avar_W2, avar_b2.reshape(1, 128),
        attr_eps0)

    ones_d = jnp.ones((D, 128), f32)
    a2 = jnp.zeros((D, 128), f32)
    a2 = a2.at[:, 0].set(dec_al)
    a2 = a2.at[:, 1].set(dec_ar)
    h, ee = _k67(z_u, xt, z_a, ones_d, wt, a2)
    elt = ee[:, 0:1].T
    out_at = _k8(ee, elt, graph, h, dec_b.reshape(1, D))

    node_mu0 = m_all[:, :HID]
    node_mu1 = m_all[:, HID:2 * HID]
    node_logv = m_all[:, 2 * HID:]

    merged_node_mu = jnp.stack([node_mu1, node_mu0], axis=1)[:, None, :, :]
    merged_node_sigma = jnp.repeat(sig_n[:, None, None, :], 2, axis=2)
    merged_node_z = jnp.repeat(z_u[:, None, None, :], 2, axis=2)
    node_logv_iw = node_logv[:, None, :]
    node_z_iw = z_u[:, None, :]

    merged_attr_mu = jnp.stack([am1, am0], axis=1)[:, None, :, :]
    merged_attr_sigma = jnp.repeat(asig[:, None, None, :], 2, axis=2)
    merged_attr_z = jnp.repeat(z_a[:, None, None, :], 2, axis=2)
    attr_logv_iw = alv[:, None, :]
    attr_z_iw = z_a[:, None, :]

    reconstruct_node_logits = links[:, :, None]
    reconstruct_attr_logits = out_at.T[:, :, None]

    return (merged_node_mu, merged_node_sigma, merged_node_z, node_logv_iw,
            node_z_iw, merged_attr_mu, merged_attr_sigma, merged_attr_z,
            attr_logv_iw, attr_z_iw, reconstruct_node_logits,
            reconstruct_attr_logits, node_mu0, am0)


# speedup vs baseline: 2.3690x; 1.0153x over previous
import jax
import jax.numpy as jnp
from jax.experimental import pallas as pl
from jax.experimental.pallas import tpu as pltpu

N = 2708
D = 1433
NOISE = 5
HID = 128
OUT = 128
F32 = jnp.float32

BM = 256
NBLK = 11
DBLK = 6
HBN = 512
HJ = 3
GAT_BM = 256
GAT_BN = 1408
GAT_NJ = 2


def _rng_consts():
    rk = jax.random.key(7)
    r = jax.random.split(rk, 4)
    node_noise = jax.random.bernoulli(r[0], 0.5, (N, 2, NOISE)).astype(F32)
    attr_noise = jax.random.bernoulli(r[1], 0.5, (D, 2, NOISE)).astype(F32)
    node_eps0 = jax.random.normal(r[2], (N, 1, OUT), dtype=F32)[:, 0, :]
    attr_eps0 = jax.random.normal(r[3], (D, 1, 128), dtype=F32)[:, 0, :]
    return node_noise, attr_noise, node_eps0, attr_eps0


try:
    _RNG_CONSTS = tuple(jax.device_get(t) for t in _rng_consts())
except Exception:
    _RNG_CONSTS = None


def _get_rng_consts():
    if _RNG_CONSTS is not None:
        return tuple(jnp.asarray(t) for t in _RNG_CONSTS)
    return _rng_consts()


def _dot(a, b):
    return jnp.dot(a, b, preferred_element_type=F32)


def _dot0(a, b):
    return jax.lax.dot_general(a, b, (((0,), (0,)), ((), ())),
                               preferred_element_type=F32)


def _dot1(a, b):
    return jax.lax.dot_general(a, b, (((1,), (1,)), ((), ())),
                               preferred_element_type=F32)


def _k1_body(xt_ref, w_ref, nn0_ref, nn1_ref, wn_ref, b1_ref, b1v_ref, o_ref):
    acc = _dot0(xt_ref[...], w_ref[...])
    xa = acc[:, :HID] + b1_ref[...]
    g1 = acc[:, HID:] + b1v_ref[...]
    h0 = xa + _dot(nn0_ref[...], wn_ref[...])
    h1 = xa + _dot(nn1_ref[...], wn_ref[...])
    o_ref[...] = jnp.concatenate([h0, h1, g1], axis=1)


def _k1(xt, wcat, nn0, nn1, wn, b1, b1v):
    return pl.pallas_call(
        _k1_body,
        grid=(NBLK,),
        in_specs=[
            pl.BlockSpec((D, BM), lambda i: (0, i)),
            pl.BlockSpec((D, 2 * HID), lambda i: (0, 0)),
            pl.BlockSpec((BM, NOISE), lambda i: (i, 0)),
            pl.BlockSpec((BM, NOISE), lambda i: (i, 0)),
            pl.BlockSpec((NOISE, HID), lambda i: (0, 0)),
            pl.BlockSpec((1, HID), lambda i: (0, 0)),
            pl.BlockSpec((1, HID), lambda i: (0, 0)),
        ],
        out_specs=pl.BlockSpec((BM, 3 * HID), lambda i: (i, 0)),
        out_shape=jax.ShapeDtypeStruct((N, 3 * HID), F32),
    )(xt, wcat, nn0, nn1, wn, b1, b1v)


def _k2_body(adj_ref, s1_ref, w2mu_ref, b2mu_ref, w2v_ref, b2v_ref, o_ref):
    t = _dot(adj_ref[...], s1_ref[...])
    r = jnp.maximum(t, 0.0)
    o_ref[...] = jnp.concatenate([
        _dot(r[:, :HID], w2mu_ref[...]) + b2mu_ref[...],
        _dot(r[:, HID:2 * HID], w2mu_ref[...]) + b2mu_ref[...],
        _dot(r[:, 2 * HID:], w2v_ref[...]) + b2v_ref[...],
    ], axis=1)


def _k2(adj, s1, w2mu, b2mu, w2v, b2v):
    return pl.pallas_call(
        _k2_body,
        grid=(NBLK,),
        in_specs=[
            pl.BlockSpec((BM, N), lambda i: (i, 0)),
            pl.BlockSpec((N, 3 * HID), lambda i: (0, 0)),
            pl.BlockSpec((HID, HID), lambda i: (0, 0)),
            pl.BlockSpec((1, HID), lambda i: (0, 0)),
            pl.BlockSpec((HID, HID), lambda i: (0, 0)),
            pl.BlockSpec((1, HID), lambda i: (0, 0)),
        ],
        out_specs=pl.BlockSpec((BM, 3 * HID), lambda i: (i, 0)),
        out_shape=jax.ShapeDtypeStruct((N, 3 * HID), F32),
    )(adj, s1, w2mu, b2mu, w2v, b2v)


def _k3_body(adj_ref, s2_ref, eps_ref, m_ref, z_ref, sig_ref):
    m = _dot(adj_ref[...], s2_ref[...])
    m_ref[...] = m
    sig = jnp.exp(0.5 * m[:, 2 * HID:])
    sig_ref[...] = sig
    z_ref[...] = m[:, :HID] + eps_ref[...] * sig


def _k3(adj, s2, eps0):
    return pl.pallas_call(
        _k3_body,
        grid=(NBLK,),
        in_specs=[
            pl.BlockSpec((BM, N), lambda i: (i, 0)),
            pl.BlockSpec((N, 3 * HID), lambda i: (0, 0)),
            pl.BlockSpec((BM, HID), lambda i: (i, 0)),
        ],
        out_specs=[
            pl.BlockSpec((BM, 3 * HID), lambda i: (i, 0)),
            pl.BlockSpec((BM, HID), lambda i: (i, 0)),
            pl.BlockSpec((BM, HID), lambda i: (i, 0)),
        ],
        out_shape=[
            jax.ShapeDtypeStruct((N, 3 * HID), F32),
            jax.ShapeDtypeStruct((N, HID), F32),
            jax.ShapeDtypeStruct((N, HID), F32),
        ],
    )(adj, s2, eps0)


def _k4_body(xt_ref, w_ref, an0_ref, an1_ref, wan_ref, b1_ref, b1v_ref,
             w2mu_ref, b2mu_ref, w2v_ref, b2v_ref, aeps_ref,
             m0_ref, m1_ref, lv_ref, sig_ref, za_ref):
    a = _dot(xt_ref[...], w_ref[...])
    base = a[:, :HID] + b1_ref[...]
    n0 = _dot(an0_ref[...], wan_ref[...])
    n1 = _dot(an1_ref[...], wan_ref[...])
    u0 = jnp.maximum(base + n0, 0.0)
    u1 = jnp.maximum(base + n1, 0.0)
    v = jnp.maximum(a[:, HID:] + b1v_ref[...], 0.0)
    m0 = _dot(u0, w2mu_ref[...]) + b2mu_ref[...]
    m1 = _dot(u1, w2mu_ref[...]) + b2mu_ref[...]
    lv = _dot(v, w2v_ref[...]) + b2v_ref[...]
    sig = jnp.exp(0.5 * lv)
    m0_ref[...] = m0
    m1_ref[...] = m1
    lv_ref[...] = lv
    sig_ref[...] = sig
    za_ref[...] = m0 + aeps_ref[...] * sig


def _k4(xt, wacat, an0, an1, wan, ab1, ab1v, w2mu, b2mu, w2v, b2v, aeps0):
    spec_row = pl.BlockSpec((BM, 128), lambda i: (i, 0))
    spec_n = pl.BlockSpec((BM, NOISE), lambda i: (i, 0))
    spec_w = pl.BlockSpec((128, 128), lambda i: (0, 0))
    spec_wn = pl.BlockSpec((NOISE, 128), lambda i: (0, 0))
    spec_b = pl.BlockSpec((1, 128), lambda i: (0, 0))
    return pl.pallas_call(
        _k4_body,
        grid=(DBLK,),
        in_specs=[
            pl.BlockSpec((BM, N), lambda i: (i, 0)),
            pl.BlockSpec((N, 2 * HID), lambda i: (0, 0)),
            spec_n, spec_n, spec_wn, spec_b, spec_b,
            spec_w, spec_b, spec_w, spec_b, spec_row,
        ],
        out_specs=[spec_row] * 5,
        out_shape=[jax.ShapeDtypeStruct((D, HID), F32)] * 5,
    )(xt, wacat, an0, an1, wan, ab1, ab1v, w2mu, b2mu, w2v, b2v, aeps0)


def _k5_body(zb_ref, zall_ref, o_ref):
    o_ref[...] = _dot1(zb_ref[...], zall_ref[...])


def _k5(z_u):
    return pl.pallas_call(
        _k5_body,
        grid=(NBLK,),
        in_specs=[
            pl.BlockSpec((BM, HID), lambda i: (i, 0)),
            pl.BlockSpec((N, HID), lambda i: (0, 0)),
        ],
        out_specs=pl.BlockSpec((BM, N), lambda i: (i, 0)),
        out_shape=jax.ShapeDtypeStruct((N, N), F32),
    )(z_u, z_u)


DA = 1536


def _k67_body(zb_ref, xt_ref, za_ref, ones_ref, wt_ref, ecol_ref,
              al_ref, ar_ref, h_ref, er_ref, elt_ref):
    zb = zb_ref[...]
    xt = xt_ref[...]
    xz = _dot0(xt, za_ref[...])
    rs = _dot0(jnp.abs(xt), ones_ref[...])
    fine = xz / jnp.maximum(rs, 1e-12)
    wt = wt_ref[...]
    h = _dot1(zb, wt[:, :HID]) + _dot1(fine, wt[:, HID:]) + ecol_ref[...]
    h_ref[...] = h.astype(jnp.bfloat16)
    er_ref[...] = _dot1(h, ar_ref[...])
    elt_ref[...] = _dot1(al_ref[...], h)


def _k67(z_u, xt, za, ones_d, wt_ext, ecol, al, ar):
    return pl.pallas_call(
        _k67_body,
        grid=(NBLK,),
        in_specs=[
            pl.BlockSpec((BM, HID), lambda i: (i, 0)),
            pl.BlockSpec((D, BM), lambda i: (0, i)),
            pl.BlockSpec((D, HID), lambda i: (0, 0)),
            pl.BlockSpec((D, 128), lambda i: (0, 0)),
            pl.BlockSpec((DA, 2 * HID), lambda i: (0, 0)),
            pl.BlockSpec((1, DA), lambda i: (0, 0)),
            pl.BlockSpec((1, DA), lambda i: (0, 0)),
            pl.BlockSpec((1, DA), lambda i: (0, 0)),
        ],
        out_specs=[
            pl.BlockSpec((BM, DA), lambda i: (i, 0)),
            pl.BlockSpec((BM, 1), lambda i: (i, 0)),
            pl.BlockSpec((1, BM), lambda i: (0, i)),
        ],
        out_shape=[
            jax.ShapeDtypeStruct((N, DA), jnp.bfloat16),
            jax.ShapeDtypeStruct((N, 1), F32),
            jax.ShapeDtypeStruct((1, N), F32),
        ],
    )(z_u, xt, za, ones_d, wt_ext, ecol, al, ar)


def _k8_body(er_ref, elt_ref, adj_ref, h_ref, b_ref, o_ref):
    e = er_ref[...] + elt_ref[...]
    e = jnp.maximum(e, 0.2 * e)
    e = jnp.where(adj_ref[...] > 0, e, -1e9)
    m = jnp.max(e, axis=1, keepdims=True)
    p = jnp.exp(e - m).astype(jnp.bfloat16)
    po = _dot(p, h_ref[...])
    out = po[:, :D] / po[:, D:D + 1] + b_ref[...]
    o_ref[...] = jnp.transpose(out)


def _k8(er, elt, adj, h, decb):
    return pl.pallas_call(
        _k8_body,
        grid=(NBLK,),
        in_specs=[
            pl.BlockSpec((GAT_BM, 1), lambda i: (i, 0)),
            pl.BlockSpec((1, N), lambda i: (0, 0)),
            pl.BlockSpec((GAT_BM, N), lambda i: (i, 0)),
            pl.BlockSpec((N, DA), lambda i: (0, 0)),
            pl.BlockSpec((1, D), lambda i: (0, 0)),
        ],
        out_specs=pl.BlockSpec((D, GAT_BM), lambda i: (0, i)),
        out_shape=jax.ShapeDtypeStruct((D, N), F32),
    )(er, elt, adj, h, decb)


def kernel(graph, x, nmu_W1, nmu_b1, nmu_W2, nmu_b2, nvar_W1, nvar_b1,
           nvar_W2, nvar_b2, amu_W1, amu_b1, amu_W2, amu_b2, avar_W1,
           avar_b1, avar_W2, avar_b2, dec_W, dec_al, dec_ar, dec_b):
    f32 = F32
    node_noise, attr_noise, node_eps0, attr_eps0 = _get_rng_consts()
    nn0 = node_noise[:, 0, :]
    nn1 = node_noise[:, 1, :]
    an0 = attr_noise[:, 0, :]
    an1 = attr_noise[:, 1, :]

    xt = x.T
    wt = dec_W.T

    wcat = jnp.concatenate([nmu_W1[NOISE:], nvar_W1], axis=1)
    wn = nmu_W1[:NOISE]
    b1 = nmu_b1.reshape(1, HID)
    b1v = nvar_b1.reshape(1, HID)

    wacat = jnp.concatenate([amu_W1[NOISE:], avar_W1], axis=1)
    wan = amu_W1[:NOISE]

    s1 = _k1(xt, wcat, nn0, nn1, wn, b1, b1v)
    s2 = _k2(graph, s1, nmu_W2, nmu_b2.reshape(1, HID),
             nvar_W2, nvar_b2.reshape(1, HID))
    m_all, z_u, sig_n = _k3(graph, s2, node_eps0)

    links = _k5(z_u)

    am0, am1, alv, asig, z_a = _k4(
        xt, wacat, an0, an1, wan,
        amu_b1.reshape(1, 128), avar_b1.reshape(1, 128),
        amu_W2, amu_b2.reshape(1, 128), avar_W2, avar_b2.reshape(1, 128),
        attr_eps0)

    ones_d = jnp.ones((D, 128), f32)
    wt_ext = jnp.pad(wt, ((0, DA - D), (0, 0)))
    ecol = jnp.zeros((1, DA), f32).at[0, D].set(1.0)
    al = jnp.pad(dec_al, (0, DA - D)).reshape(1, DA)
    ar = jnp.pad(dec_ar, (0, DA - D)).reshape(1, DA)
    h, er, elt = _k67(z_u, xt, z_a, ones_d, wt_ext, ecol, al, ar)
    out_at = _k8(er, elt, graph, h, dec_b.reshape(1, D))

    node_mu0 = m_all[:, :HID]
    node_mu1 = m_all[:, HID:2 * HID]
    node_logv = m_all[:, 2 * HID:]

    merged_node_mu = jnp.stack([node_mu1, node_mu0], axis=1)[:, None, :, :]
    merged_node_sigma = jnp.repeat(sig_n[:, None, None, :], 2, axis=2)
    merged_node_z = jnp.repeat(z_u[:, None, None, :], 2, axis=2)
    node_logv_iw = node_logv[:, None, :]
    node_z_iw = z_u[:, None, :]

    merged_attr_mu = jnp.stack([am1, am0], axis=1)[:, None, :, :]
    merged_attr_sigma = jnp.repeat(asig[:, None, None, :], 2, axis=2)
    merged_attr_z = jnp.repeat(z_a[:, None, None, :], 2, axis=2)
    attr_logv_iw = alv[:, None, :]
    attr_z_iw = z_a[:, None, :]

    reconstruct_node_logits = links[:, :, None]
    reconstruct_attr_logits = out_at.T[:, :, None]

    return (merged_node_mu, merged_node_sigma, merged_node_z, node_logv_iw,
            node_z_iw, merged_attr_mu, merged_attr_sigma, merged_attr_z,
            attr_logv_iw, attr_z_iw, reconstruct_node_logits,
            reconstruct_attr_logits, node_mu0, am0)


# speedup vs baseline: 2.5231x; 1.0650x over previous
import jax
import jax.numpy as jnp
from jax.experimental import pallas as pl
from jax.experimental.pallas import tpu as pltpu

N = 2708
D = 1433
NOISE = 5
HID = 128
OUT = 128
F32 = jnp.float32

BM = 256
NBLK = 11
DBLK = 6
HBN = 512
HJ = 3
GAT_BM = 256
GAT_BN = 1408
GAT_NJ = 2


def _rng_consts():
    rk = jax.random.key(7)
    r = jax.random.split(rk, 4)
    node_noise = jax.random.bernoulli(r[0], 0.5, (N, 2, NOISE)).astype(F32)
    attr_noise = jax.random.bernoulli(r[1], 0.5, (D, 2, NOISE)).astype(F32)
    node_eps0 = jax.random.normal(r[2], (N, 1, OUT), dtype=F32)[:, 0, :]
    attr_eps0 = jax.random.normal(r[3], (D, 1, 128), dtype=F32)[:, 0, :]
    return node_noise, attr_noise, node_eps0, attr_eps0


try:
    _RNG_CONSTS = tuple(jax.device_get(t) for t in _rng_consts())
except Exception:
    _RNG_CONSTS = None


def _get_rng_consts():
    if _RNG_CONSTS is not None:
        return tuple(jnp.asarray(t) for t in _RNG_CONSTS)
    return _rng_consts()


def _dot(a, b):
    return jnp.dot(a, b, preferred_element_type=F32)


def _dot0(a, b):
    return jax.lax.dot_general(a, b, (((0,), (0,)), ((), ())),
                               preferred_element_type=F32)


def _dot1(a, b):
    return jax.lax.dot_general(a, b, (((1,), (1,)), ((), ())),
                               preferred_element_type=F32)


def _k1_body(xt_ref, w_ref, nn0_ref, nn1_ref, wn_ref, b1_ref, b1v_ref, o_ref):
    acc = _dot0(xt_ref[...], w_ref[...])
    xa = acc[:, :HID] + b1_ref[...]
    g1 = acc[:, HID:] + b1v_ref[...]
    h0 = xa + _dot(nn0_ref[...], wn_ref[...])
    h1 = xa + _dot(nn1_ref[...], wn_ref[...])
    o_ref[...] = jnp.concatenate([h0, h1, g1], axis=1)


def _k1(xt, wcat, nn0, nn1, wn, b1, b1v):
    return pl.pallas_call(
        _k1_body,
        grid=(NBLK,),
        in_specs=[
            pl.BlockSpec((D, BM), lambda i: (0, i)),
            pl.BlockSpec((D, 2 * HID), lambda i: (0, 0)),
            pl.BlockSpec((BM, NOISE), lambda i: (i, 0)),
            pl.BlockSpec((BM, NOISE), lambda i: (i, 0)),
            pl.BlockSpec((NOISE, HID), lambda i: (0, 0)),
            pl.BlockSpec((1, HID), lambda i: (0, 0)),
            pl.BlockSpec((1, HID), lambda i: (0, 0)),
        ],
        out_specs=pl.BlockSpec((BM, 3 * HID), lambda i: (i, 0)),
        out_shape=jax.ShapeDtypeStruct((N, 3 * HID), F32),
    )(xt, wcat, nn0, nn1, wn, b1, b1v)


def _k2_body(adj_ref, s1_ref, w2mu_ref, b2mu_ref, w2v_ref, b2v_ref, o_ref):
    t = _dot(adj_ref[...], s1_ref[...])
    r = jnp.maximum(t, 0.0)
    o_ref[...] = jnp.concatenate([
        _dot(r[:, :HID], w2mu_ref[...]) + b2mu_ref[...],
        _dot(r[:, HID:2 * HID], w2mu_ref[...]) + b2mu_ref[...],
        _dot(r[:, 2 * HID:], w2v_ref[...]) + b2v_ref[...],
    ], axis=1)


def _k2(adj, s1, w2mu, b2mu, w2v, b2v):
    return pl.pallas_call(
        _k2_body,
        grid=(NBLK,),
        in_specs=[
            pl.BlockSpec((BM, N), lambda i: (i, 0)),
            pl.BlockSpec((N, 3 * HID), lambda i: (0, 0)),
            pl.BlockSpec((HID, HID), lambda i: (0, 0)),
            pl.BlockSpec((1, HID), lambda i: (0, 0)),
            pl.BlockSpec((HID, HID), lambda i: (0, 0)),
            pl.BlockSpec((1, HID), lambda i: (0, 0)),
        ],
        out_specs=pl.BlockSpec((BM, 3 * HID), lambda i: (i, 0)),
        out_shape=jax.ShapeDtypeStruct((N, 3 * HID), F32),
    )(adj, s1, w2mu, b2mu, w2v, b2v)


def _k3_body(adj_ref, s2_ref, eps_ref, m_ref, z_ref, sig_ref):
    m = _dot(adj_ref[...], s2_ref[...])
    m_ref[...] = m
    sig = jnp.exp(0.5 * m[:, 2 * HID:])
    sig_ref[...] = sig
    z_ref[...] = m[:, :HID] + eps_ref[...] * sig


def _k3(adj, s2, eps0):
    return pl.pallas_call(
        _k3_body,
        grid=(NBLK,),
        in_specs=[
            pl.BlockSpec((BM, N), lambda i: (i, 0)),
            pl.BlockSpec((N, 3 * HID), lambda i: (0, 0)),
            pl.BlockSpec((BM, HID), lambda i: (i, 0)),
        ],
        out_specs=[
            pl.BlockSpec((BM, 3 * HID), lambda i: (i, 0)),
            pl.BlockSpec((BM, HID), lambda i: (i, 0)),
            pl.BlockSpec((BM, HID), lambda i: (i, 0)),
        ],
        out_shape=[
            jax.ShapeDtypeStruct((N, 3 * HID), F32),
            jax.ShapeDtypeStruct((N, HID), F32),
            jax.ShapeDtypeStruct((N, HID), F32),
        ],
    )(adj, s2, eps0)


def _k4_body(xt_ref, w_ref, an0_ref, an1_ref, wan_ref, b1_ref, b1v_ref,
             w2mu_ref, b2mu_ref, w2v_ref, b2v_ref, aeps_ref,
             m0_ref, m1_ref, lv_ref, sig_ref, za_ref):
    a = _dot(xt_ref[...], w_ref[...])
    base = a[:, :HID] + b1_ref[...]
    n0 = _dot(an0_ref[...], wan_ref[...])
    n1 = _dot(an1_ref[...], wan_ref[...])
    u0 = jnp.maximum(base + n0, 0.0)
    u1 = jnp.maximum(base + n1, 0.0)
    v = jnp.maximum(a[:, HID:] + b1v_ref[...], 0.0)
    m0 = _dot(u0, w2mu_ref[...]) + b2mu_ref[...]
    m1 = _dot(u1, w2mu_ref[...]) + b2mu_ref[...]
    lv = _dot(v, w2v_ref[...]) + b2v_ref[...]
    sig = jnp.exp(0.5 * lv)
    m0_ref[...] = m0
    m1_ref[...] = m1
    lv_ref[...] = lv
    sig_ref[...] = sig
    za_ref[...] = m0 + aeps_ref[...] * sig


def _k4(xt, wacat, an0, an1, wan, ab1, ab1v, w2mu, b2mu, w2v, b2v, aeps0):
    spec_row = pl.BlockSpec((BM, 128), lambda i: (i, 0))
    spec_n = pl.BlockSpec((BM, NOISE), lambda i: (i, 0))
    spec_w = pl.BlockSpec((128, 128), lambda i: (0, 0))
    spec_wn = pl.BlockSpec((NOISE, 128), lambda i: (0, 0))
    spec_b = pl.BlockSpec((1, 128), lambda i: (0, 0))
    return pl.pallas_call(
        _k4_body,
        grid=(DBLK,),
        in_specs=[
            pl.BlockSpec((BM, N), lambda i: (i, 0)),
            pl.BlockSpec((N, 2 * HID), lambda i: (0, 0)),
            spec_n, spec_n, spec_wn, spec_b, spec_b,
            spec_w, spec_b, spec_w, spec_b, spec_row,
        ],
        out_specs=[spec_row] * 5,
        out_shape=[jax.ShapeDtypeStruct((D, HID), F32)] * 5,
    )(xt, wacat, an0, an1, wan, ab1, ab1v, w2mu, b2mu, w2v, b2v, aeps0)


def _k5_body(zb_ref, zall_ref, o_ref):
    o_ref[...] = _dot1(zb_ref[...], zall_ref[...])


def _k5(z_u):
    return pl.pallas_call(
        _k5_body,
        grid=(NBLK,),
        in_specs=[
            pl.BlockSpec((BM, HID), lambda i: (i, 0)),
            pl.BlockSpec((N, HID), lambda i: (0, 0)),
        ],
        out_specs=pl.BlockSpec((BM, N), lambda i: (i, 0)),
        out_shape=jax.ShapeDtypeStruct((N, N), F32),
    )(z_u, z_u)


DA = 1536


def _k67_body(zb_ref, xt_ref, za_ref, ones_ref, wt_ref, ecol_ref,
              al_ref, ar_ref, h_ref, er_ref, elt_ref):
    zb = zb_ref[...]
    xt = xt_ref[...]
    xz = _dot0(xt, za_ref[...])
    rs = _dot0(jnp.abs(xt), ones_ref[...])
    fine = xz / jnp.maximum(rs, 1e-12)
    wt = wt_ref[...]
    h = _dot1(zb, wt[:, :HID]) + _dot1(fine, wt[:, HID:]) + ecol_ref[...]
    h_ref[...] = h.astype(jnp.bfloat16)
    er_ref[...] = _dot1(h, ar_ref[...])
    elt_ref[...] = _dot1(al_ref[...], h)


def _k67(z_u, xt, za, ones_d, wt_ext, ecol, al, ar):
    return pl.pallas_call(
        _k67_body,
        grid=(NBLK,),
        in_specs=[
            pl.BlockSpec((BM, HID), lambda i: (i, 0)),
            pl.BlockSpec((D, BM), lambda i: (0, i)),
            pl.BlockSpec((D, HID), lambda i: (0, 0)),
            pl.BlockSpec((D, 128), lambda i: (0, 0)),
            pl.BlockSpec((DA, 2 * HID), lambda i: (0, 0)),
            pl.BlockSpec((1, DA), lambda i: (0, 0)),
            pl.BlockSpec((1, DA), lambda i: (0, 0)),
            pl.BlockSpec((1, DA), lambda i: (0, 0)),
        ],
        out_specs=[
            pl.BlockSpec((BM, DA), lambda i: (i, 0)),
            pl.BlockSpec((BM, 1), lambda i: (i, 0)),
            pl.BlockSpec((1, BM), lambda i: (0, i)),
        ],
        out_shape=[
            jax.ShapeDtypeStruct((N, DA), jnp.bfloat16),
            jax.ShapeDtypeStruct((N, 1), F32),
            jax.ShapeDtypeStruct((1, N), F32),
        ],
    )(z_u, xt, za, ones_d, wt_ext, ecol, al, ar)


def _k8_body(er_ref, elt_ref, adj_ref, h_ref, b_ref, o_ref):
    e = er_ref[...] + elt_ref[...]
    e = jnp.maximum(e, 0.2 * e)
    e = jnp.where(adj_ref[...] > 0, e, -1e9)
    m = jnp.max(e, axis=1, keepdims=True)
    p = jnp.exp(e - m).astype(jnp.bfloat16)
    po = _dot(p, h_ref[...])
    out = po[:, :D] / po[:, D:D + 1] + b_ref[...]
    o_ref[...] = jnp.transpose(out)


def _k8(er, elt, adj, h, decb):
    return pl.pallas_call(
        _k8_body,
        grid=(NBLK,),
        in_specs=[
            pl.BlockSpec((GAT_BM, 1), lambda i: (i, 0)),
            pl.BlockSpec((1, N), lambda i: (0, 0)),
            pl.BlockSpec((GAT_BM, N), lambda i: (i, 0)),
            pl.BlockSpec((N, DA), lambda i: (0, 0)),
            pl.BlockSpec((1, D), lambda i: (0, 0)),
        ],
        out_specs=pl.BlockSpec((D, GAT_BM), lambda i: (0, i)),
        out_shape=jax.ShapeDtypeStruct((D, N), F32),
    )(er, elt, adj, h, decb)


def kernel(graph, x, nmu_W1, nmu_b1, nmu_W2, nmu_b2, nvar_W1, nvar_b1,
           nvar_W2, nvar_b2, amu_W1, amu_b1, amu_W2, amu_b2, avar_W1,
           avar_b1, avar_W2, avar_b2, dec_W, dec_al, dec_ar, dec_b):
    f32 = F32
    node_noise, attr_noise, node_eps0, attr_eps0 = _get_rng_consts()
    nn0 = node_noise[:, 0, :]
    nn1 = node_noise[:, 1, :]
    an0 = attr_noise[:, 0, :]
    an1 = attr_noise[:, 1, :]

    xt = x.T
    wt = dec_W.T

    wcat = jnp.concatenate([nmu_W1[NOISE:], nvar_W1], axis=1)
    wn = nmu_W1[:NOISE]
    b1 = nmu_b1.reshape(1, HID)
    b1v = nvar_b1.reshape(1, HID)

    wacat = jnp.concatenate([amu_W1[NOISE:], avar_W1], axis=1)
    wan = amu_W1[:NOISE]

    s1 = _k1(xt, wcat, nn0, nn1, wn, b1, b1v)
    s2 = _k2(graph, s1, nmu_W2, nmu_b2.reshape(1, HID),
             nvar_W2, nvar_b2.reshape(1, HID))
    m_all, z_u, sig_n = _k3(graph, s2, node_eps0)

    links = _k5(z_u)

    am0, am1, alv, asig, z_a = _k4(
        xt, wacat, an0, an1, wan,
        amu_b1.reshape(1, 128), avar_b1.reshape(1, 128),
        amu_W2, amu_b2.reshape(1, 128), avar_W2, avar_b2.reshape(1, 128),
        attr_eps0)

    ones_d = jnp.ones((D, 128), f32)
    wt_ext = jnp.pad(wt, ((0, DA - D), (0, 0)))
    ecol = jnp.zeros((1, DA), f32).at[0, D].set(1.0)
    al = jnp.pad(dec_al, (0, DA - D)).reshape(1, DA)
    ar = jnp.pad(dec_ar, (0, DA - D)).reshape(1, DA)
    h, er, elt = _k67(z_u, xt, z_a, ones_d, wt_ext, ecol, al, ar)
    out_at = _k8(er, elt, graph, h, dec_b.reshape(1, D))

    node_mu0 = m_all[:, :HID]
    node_mu1 = m_all[:, HID:2 * HID]
    node_logv = m_all[:, 2 * HID:]

    merged_node_mu = jnp.stack([node_mu1, node_mu0], axis=1)[:, None, :, :]
    sig4 = sig_n[:, None, None, :]
    z4 = z_u[:, None, None, :]
    merged_node_sigma = jnp.concatenate([sig4, sig4], axis=2)
    merged_node_z = jnp.concatenate([z4, z4], axis=2)
    node_logv_iw = node_logv[:, None, :]
    node_z_iw = z_u[:, None, :]

    merged_attr_mu = jnp.stack([am1, am0], axis=1)[:, None, :, :]
    asig4 = asig[:, None, None, :]
    za4 = z_a[:, None, None, :]
    merged_attr_sigma = jnp.concatenate([asig4, asig4], axis=2)
    merged_attr_z = jnp.concatenate([za4, za4], axis=2)
    attr_logv_iw = alv[:, None, :]
    attr_z_iw = z_a[:, None, :]

    reconstruct_node_logits = links[:, :, None]
    reconstruct_attr_logits = out_at.T[:, :, None]

    return (merged_node_mu, merged_node_sigma, merged_node_z, node_logv_iw,
            node_z_iw, merged_attr_mu, merged_attr_sigma, merged_attr_z,
            attr_logv_iw, attr_z_iw, reconstruct_node_logits,
            reconstruct_attr_logits, node_mu0, am0)


# speedup vs baseline: 2.6745x; 1.0600x over previous
import jax
import jax.numpy as jnp
from jax.experimental import pallas as pl
from jax.experimental.pallas import tpu as pltpu

N = 2708
D = 1433
NOISE = 5
HID = 128
OUT = 128
F32 = jnp.float32

BM = 256
NBLK = 11
DBLK = 6
HBN = 512
HJ = 3
GAT_BM = 256
GAT_BN = 1408
GAT_NJ = 2


def _rng_consts():
    rk = jax.random.key(7)
    r = jax.random.split(rk, 4)
    node_noise = jax.random.bernoulli(r[0], 0.5, (N, 2, NOISE)).astype(F32)
    attr_noise = jax.random.bernoulli(r[1], 0.5, (D, 2, NOISE)).astype(F32)
    node_eps0 = jax.random.normal(r[2], (N, 1, OUT), dtype=F32)[:, 0, :]
    attr_eps0 = jax.random.normal(r[3], (D, 1, 128), dtype=F32)[:, 0, :]
    return node_noise, attr_noise, node_eps0, attr_eps0


try:
    _RNG_CONSTS = tuple(jax.device_get(t) for t in _rng_consts())
except Exception:
    _RNG_CONSTS = None


def _get_rng_consts():
    if _RNG_CONSTS is not None:
        return tuple(jnp.asarray(t) for t in _RNG_CONSTS)
    return _rng_consts()


def _dot(a, b):
    return jnp.dot(a, b, preferred_element_type=F32)


def _dot0(a, b):
    return jax.lax.dot_general(a, b, (((0,), (0,)), ((), ())),
                               preferred_element_type=F32)


def _dot1(a, b):
    return jax.lax.dot_general(a, b, (((1,), (1,)), ((), ())),
                               preferred_element_type=F32)


def _k1_body(xt_ref, w_ref, nn0_ref, nn1_ref, wn_ref, b1_ref, b1v_ref, o_ref):
    acc = _dot0(xt_ref[...], w_ref[...])
    xa = acc[:, :HID] + b1_ref[...]
    g1 = acc[:, HID:] + b1v_ref[...]
    h0 = xa + _dot(nn0_ref[...], wn_ref[...])
    h1 = xa + _dot(nn1_ref[...], wn_ref[...])
    o_ref[...] = jnp.concatenate([h0, h1, g1], axis=1)


def _k1(xt, wcat, nn0, nn1, wn, b1, b1v):
    return pl.pallas_call(
        _k1_body,
        grid=(NBLK,),
        in_specs=[
            pl.BlockSpec((D, BM), lambda i: (0, i)),
            pl.BlockSpec((D, 2 * HID), lambda i: (0, 0)),
            pl.BlockSpec((BM, NOISE), lambda i: (i, 0)),
            pl.BlockSpec((BM, NOISE), lambda i: (i, 0)),
            pl.BlockSpec((NOISE, HID), lambda i: (0, 0)),
            pl.BlockSpec((1, HID), lambda i: (0, 0)),
            pl.BlockSpec((1, HID), lambda i: (0, 0)),
        ],
        out_specs=pl.BlockSpec((BM, 3 * HID), lambda i: (i, 0)),
        out_shape=jax.ShapeDtypeStruct((N, 3 * HID), F32),
    )(xt, wcat, nn0, nn1, wn, b1, b1v)


def _k2_body(adj_ref, s1_ref, w2mu_ref, b2mu_ref, w2v_ref, b2v_ref, o_ref):
    t = _dot(adj_ref[...], s1_ref[...])
    r = jnp.maximum(t, 0.0)
    o_ref[...] = jnp.concatenate([
        _dot(r[:, :HID], w2mu_ref[...]) + b2mu_ref[...],
        _dot(r[:, HID:2 * HID], w2mu_ref[...]) + b2mu_ref[...],
        _dot(r[:, 2 * HID:], w2v_ref[...]) + b2v_ref[...],
    ], axis=1)


def _k2(adj, s1, w2mu, b2mu, w2v, b2v):
    return pl.pallas_call(
        _k2_body,
        grid=(NBLK,),
        in_specs=[
            pl.BlockSpec((BM, N), lambda i: (i, 0)),
            pl.BlockSpec((N, 3 * HID), lambda i: (0, 0)),
            pl.BlockSpec((HID, HID), lambda i: (0, 0)),
            pl.BlockSpec((1, HID), lambda i: (0, 0)),
            pl.BlockSpec((HID, HID), lambda i: (0, 0)),
            pl.BlockSpec((1, HID), lambda i: (0, 0)),
        ],
        out_specs=pl.BlockSpec((BM, 3 * HID), lambda i: (i, 0)),
        out_shape=jax.ShapeDtypeStruct((N, 3 * HID), F32),
    )(adj, s1, w2mu, b2mu, w2v, b2v)


def _k3_body(adj_ref, s2_ref, eps_ref, m_ref, z_ref, sig_ref):
    m = _dot(adj_ref[...], s2_ref[...])
    m_ref[...] = m
    sig = jnp.exp(0.5 * m[:, 2 * HID:])
    sig_ref[...] = sig
    z_ref[...] = m[:, :HID] + eps_ref[...] * sig


def _k3(adj, s2, eps0):
    return pl.pallas_call(
        _k3_body,
        grid=(NBLK,),
        in_specs=[
            pl.BlockSpec((BM, N), lambda i: (i, 0)),
            pl.BlockSpec((N, 3 * HID), lambda i: (0, 0)),
            pl.BlockSpec((BM, HID), lambda i: (i, 0)),
        ],
        out_specs=[
            pl.BlockSpec((BM, 3 * HID), lambda i: (i, 0)),
            pl.BlockSpec((BM, HID), lambda i: (i, 0)),
            pl.BlockSpec((BM, HID), lambda i: (i, 0)),
        ],
        out_shape=[
            jax.ShapeDtypeStruct((N, 3 * HID), F32),
            jax.ShapeDtypeStruct((N, HID), F32),
            jax.ShapeDtypeStruct((N, HID), F32),
        ],
    )(adj, s2, eps0)


def _k4_body(xt_ref, w_ref, an0_ref, an1_ref, wan_ref, b1_ref, b1v_ref,
             w2mu_ref, b2mu_ref, w2v_ref, b2v_ref, aeps_ref,
             m0_ref, m1_ref, lv_ref, sig_ref, za_ref):
    a = _dot(xt_ref[...], w_ref[...])
    base = a[:, :HID] + b1_ref[...]
    n0 = _dot(an0_ref[...], wan_ref[...])
    n1 = _dot(an1_ref[...], wan_ref[...])
    u0 = jnp.maximum(base + n0, 0.0)
    u1 = jnp.maximum(base + n1, 0.0)
    v = jnp.maximum(a[:, HID:] + b1v_ref[...], 0.0)
    m0 = _dot(u0, w2mu_ref[...]) + b2mu_ref[...]
    m1 = _dot(u1, w2mu_ref[...]) + b2mu_ref[...]
    lv = _dot(v, w2v_ref[...]) + b2v_ref[...]
    sig = jnp.exp(0.5 * lv)
    m0_ref[...] = m0
    m1_ref[...] = m1
    lv_ref[...] = lv
    sig_ref[...] = sig
    za_ref[...] = m0 + aeps_ref[...] * sig


def _k4(xt, wacat, an0, an1, wan, ab1, ab1v, w2mu, b2mu, w2v, b2v, aeps0):
    spec_row = pl.BlockSpec((BM, 128), lambda i: (i, 0))
    spec_n = pl.BlockSpec((BM, NOISE), lambda i: (i, 0))
    spec_w = pl.BlockSpec((128, 128), lambda i: (0, 0))
    spec_wn = pl.BlockSpec((NOISE, 128), lambda i: (0, 0))
    spec_b = pl.BlockSpec((1, 128), lambda i: (0, 0))
    return pl.pallas_call(
        _k4_body,
        grid=(DBLK,),
        in_specs=[
            pl.BlockSpec((BM, N), lambda i: (i, 0)),
            pl.BlockSpec((N, 2 * HID), lambda i: (0, 0)),
            spec_n, spec_n, spec_wn, spec_b, spec_b,
            spec_w, spec_b, spec_w, spec_b, spec_row,
        ],
        out_specs=[spec_row] * 5,
        out_shape=[jax.ShapeDtypeStruct((D, HID), F32)] * 5,
    )(xt, wacat, an0, an1, wan, ab1, ab1v, w2mu, b2mu, w2v, b2v, aeps0)


def _k5_body(zb_ref, zall_ref, o_ref):
    o_ref[...] = _dot1(zb_ref[...], zall_ref[...])


def _k5(z_u):
    return pl.pallas_call(
        _k5_body,
        grid=(NBLK,),
        in_specs=[
            pl.BlockSpec((BM, HID), lambda i: (i, 0)),
            pl.BlockSpec((N, HID), lambda i: (0, 0)),
        ],
        out_specs=pl.BlockSpec((BM, N), lambda i: (i, 0)),
        out_shape=jax.ShapeDtypeStruct((N, N), F32),
    )(z_u, z_u)


def _k67_body(zb_ref, xt_ref, za_ref, ones_ref, wt_ref, alr_ref, ec_ref,
              cf_ref, el_ref, er_ref):
    zb = zb_ref[...]
    xt = xt_ref[...]
    xz = _dot0(xt, za_ref[...])
    rs = _dot0(jnp.abs(xt), ones_ref[...])
    fine = xz / jnp.maximum(rs, 1e-12)
    cf256 = jnp.concatenate([zb, fine], axis=1)
    wlr = _dot0(wt_ref[...], alr_ref[...])
    el_ref[...] = _dot(cf256, wlr[:, 0:1])
    er_ref[...] = _dot(cf256, wlr[:, 1:2])
    cf_ref[...] = jnp.concatenate(
        [cf256, jnp.broadcast_to(ec_ref[...], (BM, 128))], axis=1
    ).astype(jnp.bfloat16)


def _k67(z_u, xt, za, ones_d, wt, alr, ec):
    return pl.pallas_call(
        _k67_body,
        grid=(NBLK,),
        in_specs=[
            pl.BlockSpec((BM, HID), lambda i: (i, 0)),
            pl.BlockSpec((D, BM), lambda i: (0, i)),
            pl.BlockSpec((D, HID), lambda i: (0, 0)),
            pl.BlockSpec((D, 128), lambda i: (0, 0)),
            pl.BlockSpec((D, 2 * HID), lambda i: (0, 0)),
            pl.BlockSpec((D, 2), lambda i: (0, 0)),
            pl.BlockSpec((1, 128), lambda i: (0, 0)),
        ],
        out_specs=[
            pl.BlockSpec((BM, 3 * HID), lambda i: (i, 0)),
            pl.BlockSpec((BM, 1), lambda i: (i, 0)),
            pl.BlockSpec((BM, 1), lambda i: (i, 0)),
        ],
        out_shape=[
            jax.ShapeDtypeStruct((N, 3 * HID), jnp.bfloat16),
            jax.ShapeDtypeStruct((N, 1), F32),
            jax.ShapeDtypeStruct((N, 1), F32),
        ],
    )(z_u, xt, za, ones_d, wt, alr, ec)


def _k8_body(er_ref, elt_ref, adj_ref, cf_ref, wt_ref, b_ref, o_ref):
    e = er_ref[...] + elt_ref[...]
    e = jnp.maximum(e, 0.2 * e)
    e = jnp.where(adj_ref[...] > 0, e, -1e9)
    m = jnp.max(e, axis=1, keepdims=True)
    p = jnp.exp(e - m).astype(jnp.bfloat16)
    acf = _dot(p, cf_ref[...])
    l = acf[:, 2 * HID:2 * HID + 1]
    alpha = (acf[:, :2 * HID] / l).astype(jnp.bfloat16)
    out = _dot1(alpha, wt_ref[...]) + b_ref[...]
    o_ref[...] = jnp.transpose(out)


def _k8(er, elt, adj, cf, wt_bf, decb):
    return pl.pallas_call(
        _k8_body,
        grid=(NBLK,),
        in_specs=[
            pl.BlockSpec((GAT_BM, 1), lambda i: (i, 0)),
            pl.BlockSpec((1, N), lambda i: (0, 0)),
            pl.BlockSpec((GAT_BM, N), lambda i: (i, 0)),
            pl.BlockSpec((N, 3 * HID), lambda i: (0, 0)),
            pl.BlockSpec((D, 2 * HID), lambda i: (0, 0)),
            pl.BlockSpec((1, D), lambda i: (0, 0)),
        ],
        out_specs=pl.BlockSpec((D, GAT_BM), lambda i: (0, i)),
        out_shape=jax.ShapeDtypeStruct((D, N), F32),
    )(er, elt, adj, cf, wt_bf, decb)


def kernel(graph, x, nmu_W1, nmu_b1, nmu_W2, nmu_b2, nvar_W1, nvar_b1,
           nvar_W2, nvar_b2, amu_W1, amu_b1, amu_W2, amu_b2, avar_W1,
           avar_b1, avar_W2, avar_b2, dec_W, dec_al, dec_ar, dec_b):
    f32 = F32
    node_noise, attr_noise, node_eps0, attr_eps0 = _get_rng_consts()
    nn0 = node_noise[:, 0, :]
    nn1 = node_noise[:, 1, :]
    an0 = attr_noise[:, 0, :]
    an1 = attr_noise[:, 1, :]

    xt = x.T
    wt = dec_W.T

    wcat = jnp.concatenate([nmu_W1[NOISE:], nvar_W1], axis=1)
    wn = nmu_W1[:NOISE]
    b1 = nmu_b1.reshape(1, HID)
    b1v = nvar_b1.reshape(1, HID)

    wacat = jnp.concatenate([amu_W1[NOISE:], avar_W1], axis=1)
    wan = amu_W1[:NOISE]

    s1 = _k1(xt, wcat, nn0, nn1, wn, b1, b1v)
    s2 = _k2(graph, s1, nmu_W2, nmu_b2.reshape(1, HID),
             nvar_W2, nvar_b2.reshape(1, HID))
    m_all, z_u, sig_n = _k3(graph, s2, node_eps0)

    links = _k5(z_u)

    am0, am1, alv, asig, z_a = _k4(
        xt, wacat, an0, an1, wan,
        amu_b1.reshape(1, 128), avar_b1.reshape(1, 128),
        amu_W2, amu_b2.reshape(1, 128), avar_W2, avar_b2.reshape(1, 128),
        attr_eps0)

    ones_d = jnp.ones((D, 128), f32)
    alr = jnp.stack([dec_al, dec_ar], axis=1)
    ec = jnp.zeros((1, 128), f32).at[0, 0].set(1.0)
    cf, el, er = _k67(z_u, xt, z_a, ones_d, wt, alr, ec)
    elt = el.T
    out_at = _k8(er, elt, graph, cf, wt.astype(jnp.bfloat16),
                 dec_b.reshape(1, D))

    node_mu0 = m_all[:, :HID]
    node_mu1 = m_all[:, HID:2 * HID]
    node_logv = m_all[:, 2 * HID:]

    merged_node_mu = jnp.stack([node_mu1, node_mu0], axis=1)[:, None, :, :]
    sig4 = sig_n[:, None, None, :]
    z4 = z_u[:, None, None, :]
    merged_node_sigma = jnp.concatenate([sig4, sig4], axis=2)
    merged_node_z = jnp.concatenate([z4, z4], axis=2)
    node_logv_iw = node_logv[:, None, :]
    node_z_iw = z_u[:, None, :]

    merged_attr_mu = jnp.stack([am1, am0], axis=1)[:, None, :, :]
    asig4 = asig[:, None, None, :]
    za4 = z_a[:, None, None, :]
    merged_attr_sigma = jnp.concatenate([asig4, asig4], axis=2)
    merged_attr_z = jnp.concatenate([za4, za4], axis=2)
    attr_logv_iw = alv[:, None, :]
    attr_z_iw = z_a[:, None, :]

    reconstruct_node_logits = links[:, :, None]
    reconstruct_attr_logits = out_at.T[:, :, None]

    return (merged_node_mu, merged_node_sigma, merged_node_z, node_logv_iw,
            node_z_iw, merged_attr_mu, merged_attr_sigma, merged_attr_z,
            attr_logv_iw, attr_z_iw, reconstruct_node_logits,
            reconstruct_attr_logits, node_mu0, am0)
